# Initial kernel scaffold; baseline (speedup 1.0000x reference)
import jax
import jax.numpy as jnp
from jax.experimental import pallas as pl


def kernel(x, edge_index, edge_attr, Wq0, bq0, Wk0, bk0, Wv0, bv0, We0, Ws0, bs0, Wq1, bq1, Wk1, bk1, Wv1, bv1, We1, Ws1, bs1, Wlin, blin):
    raise NotImplementedError("write your pallas kernel here")


# speedup vs baseline: 1.4836x; 1.4836x over previous
import jax
import jax.numpy as jnp
from jax import lax
from jax.experimental import pallas as pl
from jax.experimental.pallas import tpu as pltpu
from jax.experimental.pallas import tpu_sc as plsc

_N = 10000
_E = 320000
_C = 128
_ED = 16
_HC = 64
_NC = 2
_NS = 16
_QQW = _C + _ED
_KVW = _C + _HC
_ACCW = 96
_CHUNK = 80
_EPT = _E // _NS
_NCHUNK = _EPT // _CHUNK
_NP = 10240
_RPT = _NP // _NS
_RCHUNK = 128
_INV_SQRT_C = 1.0 / float(_C) ** 0.5


def _proj_body(x_ref, w_ref, b_ref, we_ref, qq_ref, kvh_ref, s_ref):
    z = jnp.dot(x_ref[...], w_ref[...], preferred_element_type=jnp.float32)
    z = z + b_ref[...]
    q = z[:, :_C]
    qw = jnp.dot(q, we_ref[...], preferred_element_type=jnp.float32)
    qq_ref[...] = jnp.concatenate([q, qw], axis=1)
    k = z[:, _C:2 * _C]
    v = z[:, 2 * _C:3 * _C]
    kvh_ref[0] = jnp.concatenate([k, v[:, :_HC]], axis=1)
    kvh_ref[1] = jnp.concatenate([k, v[:, _HC:]], axis=1)
    s_ref[...] = z[:, 3 * _C:]


def _proj(x2, wcat_t, bcat, we):
    ind = x2.shape[1]
    blk = 2000
    return pl.pallas_call(
        _proj_body,
        grid=(_N // blk,),
        in_specs=[
            pl.BlockSpec((blk, ind), lambda i: (i, 0)),
            pl.BlockSpec((ind, 4 * _C), lambda i: (0, 0)),
            pl.BlockSpec((1, 4 * _C), lambda i: (0, 0)),
            pl.BlockSpec((_C, _ED), lambda i: (0, 0)),
        ],
        out_specs=[
            pl.BlockSpec((blk, _QQW), lambda i: (i, 0)),
            pl.BlockSpec((_NC, blk, _KVW), lambda i: (0, i, 0)),
            pl.BlockSpec((blk, _C), lambda i: (i, 0)),
        ],
        out_shape=[
            jax.ShapeDtypeStruct((_N, _QQW), jnp.float32),
            jax.ShapeDtypeStruct((_NC, _N, _KVW), jnp.float32),
            jax.ShapeDtypeStruct((_N, _C), jnp.float32),
        ],
    )(x2, wcat_t, bcat, we)


def _post_body(a0_ref, a1_ref, s_ref, wet_ref, h_ref):
    num = jnp.concatenate([a0_ref[:, :_HC], a1_ref[:, :_HC]], axis=1)
    num = num + jnp.dot(a0_ref[:, _HC:_HC + _ED], wet_ref[...],
                        preferred_element_type=jnp.float32)
    den = a0_ref[:, _HC + _ED:_HC + _ED + 1]
    h_ref[...] = num / (den + 1e-16) + s_ref[...]


def _post1(a0, a1, s, wet):
    blk = 2000
    return pl.pallas_call(
        _post_body,
        grid=(_N // blk,),
        in_specs=[
            pl.BlockSpec((blk, _ACCW), lambda i: (i, 0)),
            pl.BlockSpec((blk, _ACCW), lambda i: (i, 0)),
            pl.BlockSpec((blk, _C), lambda i: (i, 0)),
            pl.BlockSpec((_ED, _C), lambda i: (0, 0)),
        ],
        out_specs=pl.BlockSpec((blk, _C), lambda i: (i, 0)),
        out_shape=jax.ShapeDtypeStruct((_N, _C), jnp.float32),
    )(a0, a1, s, wet)


def _post2_body(a0_ref, a1_ref, s_ref, wet_ref, wlt_ref, bl_ref, o_ref):
    num = jnp.concatenate([a0_ref[:, :_HC], a1_ref[:, :_HC]], axis=1)
    num = num + jnp.dot(a0_ref[:, _HC:_HC + _ED], wet_ref[...],
                        preferred_element_type=jnp.float32)
    den = a0_ref[:, _HC + _ED:_HC + _ED + 1]
    h = num / (den + 1e-16) + s_ref[...]
    o_ref[...] = jnp.dot(h, wlt_ref[...],
                         preferred_element_type=jnp.float32) + bl_ref[...]


def _post2(a0, a1, s, wet, wlt, bl):
    blk = 2000
    return pl.pallas_call(
        _post2_body,
        grid=(_N // blk,),
        in_specs=[
            pl.BlockSpec((blk, _ACCW), lambda i: (i, 0)),
            pl.BlockSpec((blk, _ACCW), lambda i: (i, 0)),
            pl.BlockSpec((blk, _C), lambda i: (i, 0)),
            pl.BlockSpec((_ED, _C), lambda i: (0, 0)),
            pl.BlockSpec((_C, _C), lambda i: (0, 0)),
            pl.BlockSpec((1, _C), lambda i: (0, 0)),
        ],
        out_specs=pl.BlockSpec((blk, _C), lambda i: (i, 0)),
        out_shape=jax.ShapeDtypeStruct((_N, _C), jnp.float32),
    )(a0, a1, s, wet, wlt, bl)


def _sc_edge_body(qq_hbm, kvh_hbm, src_hbm, dst_hbm, ea_hbm, out_hbm,
                  src_v, dst_v, qq_v, kv_v, a_v, msg_v, zbuf, acc_sh,
                  sem1, sem2):
    cid = lax.axis_index("c")
    sid = lax.axis_index("s")
    zero16 = jnp.zeros((16,), jnp.float32)

    @pl.loop(0, _RCHUNK)
    def _zr(r):
        @pl.loop(0, _ACCW // 16)
        def _zc(c):
            zbuf[r, pl.ds(c * 16, 16)] = zero16

    @pl.loop(0, _CHUNK)
    def _zm(r):
        msg_v[r, pl.ds(_HC + _ED, 16)] = zero16

    @pl.loop(0, _RPT // _RCHUNK)
    def _zi(t):
        pltpu.sync_copy(zbuf, acc_sh.at[pl.ds(sid * _RPT + t * _RCHUNK,
                                              _RCHUNK), :])

    plsc.subcore_barrier()

    ebase = sid * _EPT
    rows0 = lax.iota(jnp.int32, 16)
    den_col = jnp.full((16,), _HC + _ED, jnp.int32)

    @pl.loop(0, _NCHUNK)
    def _chunk(g):
        base = ebase + g * _CHUNK
        pltpu.sync_copy(src_hbm.at[pl.ds(base, _CHUNK)], src_v)
        pltpu.sync_copy(dst_hbm.at[pl.ds(base, _CHUNK)], dst_v)
        pltpu.sync_copy(ea_hbm.at[pl.ds(base, _CHUNK), :], a_v)
        cp1 = pltpu.async_copy(qq_hbm.at[dst_v], qq_v, sem1)
        cp2 = pltpu.async_copy(kvh_hbm.at[cid].at[src_v], kv_v, sem2)
        cp1.wait()
        cp2.wait()

        for grp in range(_CHUNK // 16):
            rows = rows0 + (grp * 16)

            @pl.loop(0, _C, init_carry=zero16, unroll=8)
            def alpha(c, acc):
                cv = jnp.full((16,), c, jnp.int32)
                qcol = plsc.load_gather(qq_v, [rows, cv])
                kcol = plsc.load_gather(kv_v, [rows, cv])
                return acc + qcol * kcol

            @pl.loop(0, _ED, init_carry=alpha, unroll=8)
            def alpha(c, acc):
                cv = jnp.full((16,), c, jnp.int32)
                qwcol = plsc.load_gather(qq_v, [rows, cv + _C])
                acol = plsc.load_gather(a_v, [rows, cv])
                return acc + qwcol * acol

            ex = jnp.exp(alpha * _INV_SQRT_C)

            @pl.loop(0, _HC, unroll=8)
            def _msg_v_cols(c):
                cv = jnp.full((16,), c, jnp.int32)
                vcol = plsc.load_gather(kv_v, [rows, cv + _C])
                plsc.store_scatter(msg_v, [rows, cv], ex * vcol)

            @pl.loop(0, _ED, unroll=8)
            def _msg_a_cols(c):
                cv = jnp.full((16,), c, jnp.int32)
                acol = plsc.load_gather(a_v, [rows, cv])
                plsc.store_scatter(msg_v, [rows, cv + _HC], ex * acol)

            plsc.store_scatter(msg_v, [rows, den_col], ex)

        pltpu.sync_copy(msg_v, acc_sh.at[dst_v], add=True)

    plsc.subcore_barrier()

    @pl.loop(0, _RPT // _RCHUNK)
    def _wout(t):
        r0 = sid * _RPT + t * _RCHUNK
        pltpu.sync_copy(acc_sh.at[pl.ds(r0, _RCHUNK), :], zbuf)
        pltpu.sync_copy(zbuf, out_hbm.at[cid, pl.ds(r0, _RCHUNK), :])


_sc_mesh = plsc.VectorSubcoreMesh(core_axis_name="c", subcore_axis_name="s",
                                  num_cores=_NC, num_subcores=_NS)

_sc_edge = pl.kernel(
    _sc_edge_body,
    out_type=jax.ShapeDtypeStruct((_NC, _NP, _ACCW), jnp.float32),
    mesh=_sc_mesh,
    compiler_params=pltpu.CompilerParams(use_tc_tiling_on_sc=False,
                                         needs_layout_passes=False),
    scratch_types=[
        pltpu.VMEM((_CHUNK,), jnp.int32),
        pltpu.VMEM((_CHUNK,), jnp.int32),
        pltpu.VMEM((_CHUNK, _QQW), jnp.float32),
        pltpu.VMEM((_CHUNK, _KVW), jnp.float32),
        pltpu.VMEM((_CHUNK, _ED), jnp.float32),
        pltpu.VMEM((_CHUNK, _ACCW), jnp.float32),
        pltpu.VMEM((_RCHUNK, _ACCW), jnp.float32),
        pltpu.VMEM_SHARED((_NP, _ACCW), jnp.float32),
        pltpu.SemaphoreType.DMA,
        pltpu.SemaphoreType.DMA,
    ],
)


@jax.jit
def kernel(x, edge_index, edge_attr,
           Wq0, bq0, Wk0, bk0, Wv0, bv0, We0, Ws0, bs0,
           Wq1, bq1, Wk1, bk1, Wv1, bv1, We1, Ws1, bs1,
           Wlin, blin):
    src = edge_index[0]
    dst = edge_index[1]

    w0t = jnp.concatenate([Wq0, Wk0, Wv0, Ws0], axis=0).T
    b0 = jnp.concatenate([bq0, bk0, bv0, bs0]).reshape(1, -1)
    qq0, kvh0, s0 = _proj(x, w0t, b0, We0)
    parts0 = _sc_edge(qq0, kvh0, src, dst, edge_attr)
    h1 = _post1(parts0[0], parts0[1], s0, We0.T)

    x2 = jnp.concatenate([h1, x], axis=1)
    w1t = jnp.concatenate([Wq1, Wk1, Wv1, Ws1], axis=0).T
    b1 = jnp.concatenate([bq1, bk1, bv1, bs1]).reshape(1, -1)
    qq1, kvh1, s1 = _proj(x2, w1t, b1, We1)
    parts1 = _sc_edge(qq1, kvh1, src, dst, edge_attr)
    return _post2(parts1[0], parts1[1], s1, We1.T, Wlin.T,
                  blin.reshape(1, -1))


# speedup vs baseline: 2.0363x; 1.3726x over previous
import jax
import jax.numpy as jnp
from jax import lax
from jax.experimental import pallas as pl
from jax.experimental.pallas import tpu as pltpu
from jax.experimental.pallas import tpu_sc as plsc

_N = 10000
_E = 320000
_C = 128
_ED = 16
_HC = 64
_NC = 2
_NS = 16
_QQW = _C + _ED
_KVW = _C + _HC
_AH = _ED // 2
_ACCW = 80
_CHUNK = 80
_EPT = _E // _NS
_NCHUNK = _EPT // _CHUNK
_NP = 10240
_RPT = _NP // _NS
_RCHUNK = 32
_INV_SQRT_C = 1.0 / float(_C) ** 0.5


def _proj_body(x_ref, w_ref, b_ref, we_ref, qq_ref, kvh_ref, s_ref):
    z = lax.dot_general(x_ref[...], w_ref[...], (((1,), (1,)), ((), ())),
                        preferred_element_type=jnp.float32)
    z = z + b_ref[...]
    q = z[:, :_C]
    qw = jnp.dot(q, we_ref[...], preferred_element_type=jnp.float32)
    qq_ref[...] = jnp.concatenate([q, qw], axis=1)
    k = z[:, _C:2 * _C]
    v = z[:, 2 * _C:3 * _C]
    kvh_ref[0] = jnp.concatenate([k, v[:, :_HC]], axis=1)
    kvh_ref[1] = jnp.concatenate([k, v[:, _HC:]], axis=1)
    s_ref[...] = z[:, 3 * _C:]


def _proj(x2, wcat_t, bcat, we):
    ind = x2.shape[1]
    blk = 2000
    return pl.pallas_call(
        _proj_body,
        grid=(_N // blk,),
        in_specs=[
            pl.BlockSpec((blk, ind), lambda i: (i, 0)),
            pl.BlockSpec((4 * _C, ind), lambda i: (0, 0)),
            pl.BlockSpec((1, 4 * _C), lambda i: (0, 0)),
            pl.BlockSpec((_C, _ED), lambda i: (0, 0)),
        ],
        out_specs=[
            pl.BlockSpec((blk, _QQW), lambda i: (i, 0)),
            pl.BlockSpec((_NC, blk, _KVW), lambda i: (0, i, 0)),
            pl.BlockSpec((blk, _C), lambda i: (i, 0)),
        ],
        out_shape=[
            jax.ShapeDtypeStruct((_N, _QQW), jnp.float32),
            jax.ShapeDtypeStruct((_NC, _N, _KVW), jnp.float32),
            jax.ShapeDtypeStruct((_N, _C), jnp.float32),
        ],
    )(x2, wcat_t, bcat, we)


def _post_body(a0_ref, a1_ref, s_ref, wet_ref, h_ref):
    num = jnp.concatenate([a0_ref[:, :_HC], a1_ref[:, :_HC]], axis=1)
    t = jnp.concatenate([a0_ref[:, _HC:_HC + _AH],
                         a1_ref[:, _HC:_HC + _AH]], axis=1)
    num = num + lax.dot_general(t, wet_ref[...], (((1,), (1,)), ((), ())),
                                preferred_element_type=jnp.float32)
    den = a0_ref[:, _HC + _AH:_HC + _AH + 1]
    h_ref[...] = num / (den + 1e-16) + s_ref[...]


def _post1(a0, a1, s, wet):
    blk = 2000
    return pl.pallas_call(
        _post_body,
        grid=(_N // blk,),
        in_specs=[
            pl.BlockSpec((blk, _ACCW), lambda i: (i, 0)),
            pl.BlockSpec((blk, _ACCW), lambda i: (i, 0)),
            pl.BlockSpec((blk, _C), lambda i: (i, 0)),
            pl.BlockSpec((_C, _ED), lambda i: (0, 0)),
        ],
        out_specs=pl.BlockSpec((blk, _C), lambda i: (i, 0)),
        out_shape=jax.ShapeDtypeStruct((_N, _C), jnp.float32),
    )(a0, a1, s, wet)


def _post2_body(a0_ref, a1_ref, s_ref, wet_ref, wlt_ref, bl_ref, o_ref):
    num = jnp.concatenate([a0_ref[:, :_HC], a1_ref[:, :_HC]], axis=1)
    t = jnp.concatenate([a0_ref[:, _HC:_HC + _AH],
                         a1_ref[:, _HC:_HC + _AH]], axis=1)
    num = num + lax.dot_general(t, wet_ref[...], (((1,), (1,)), ((), ())),
                                preferred_element_type=jnp.float32)
    den = a0_ref[:, _HC + _AH:_HC + _AH + 1]
    h = num / (den + 1e-16) + s_ref[...]
    o_ref[...] = lax.dot_general(h, wlt_ref[...], (((1,), (1,)), ((), ())),
                                 preferred_element_type=jnp.float32)
    o_ref[...] += bl_ref[...]


def _post2(a0, a1, s, wet, wlt, bl):
    blk = 2000
    return pl.pallas_call(
        _post2_body,
        grid=(_N // blk,),
        in_specs=[
            pl.BlockSpec((blk, _ACCW), lambda i: (i, 0)),
            pl.BlockSpec((blk, _ACCW), lambda i: (i, 0)),
            pl.BlockSpec((blk, _C), lambda i: (i, 0)),
            pl.BlockSpec((_C, _ED), lambda i: (0, 0)),
            pl.BlockSpec((_C, _C), lambda i: (0, 0)),
            pl.BlockSpec((1, _C), lambda i: (0, 0)),
        ],
        out_specs=pl.BlockSpec((blk, _C), lambda i: (i, 0)),
        out_shape=jax.ShapeDtypeStruct((_N, _C), jnp.float32),
    )(a0, a1, s, wet, wlt, bl)


def _sc_edge_body(qq_hbm, kvh_hbm, ei_hbm, ea_hbm, out_hbm,
                  ei_v, da_v, qq_v, kv_v, a_v, msg_v, zbuf, acc_sh,
                  sem_q, sem_g0, sem_g1, sem_s0, sem_s1):
    cid = lax.axis_index("c")
    sid = lax.axis_index("s")
    zero16 = jnp.zeros((16,), jnp.float32)

    @pl.loop(0, _RCHUNK)
    def _zr(r):
        @pl.loop(0, _ACCW // 16)
        def _zc(c):
            zbuf[r, pl.ds(c * 16, 16)] = zero16

    for b in range(2):
        @pl.loop(0, _CHUNK)
        def _zm(r):
            msg_v[b][r, pl.ds(_HC, 16)] = zero16

    @pl.loop(0, _RPT // _RCHUNK)
    def _zi(t):
        pltpu.sync_copy(zbuf, acc_sh.at[pl.ds(sid * _RPT + t * _RCHUNK,
                                              _RCHUNK), :])

    plsc.subcore_barrier()

    crow0 = sid * _NCHUNK
    rows0 = lax.iota(jnp.int32, 16)
    den_col = jnp.full((16,), _HC + _AH, jnp.int32)
    gsem = [sem_g0, sem_g1]
    ssem = [sem_s0, sem_s1]

    def _alpha_all(b):
        exs = []
        for grp in range(_CHUNK // 16):
            rows = rows0 + (grp * 16)
            aoff = b * _CHUNK + grp * 16

            @pl.loop(0, _C, init_carry=zero16, unroll=8)
            def alpha(c, acc):
                cv = jnp.full((16,), c, jnp.int32)
                qcol = plsc.load_gather(qq_v, [rows, cv])
                kcol = plsc.load_gather(kv_v[b], [rows, cv])
                return acc + qcol * kcol

            @pl.loop(0, _ED, init_carry=alpha, unroll=8)
            def alpha(c, acc):
                cv = jnp.full((16,), c, jnp.int32)
                qwcol = plsc.load_gather(qq_v, [rows, cv + _C])
                acol = a_v[c, pl.ds(aoff, 16)]
                return acc + qwcol * acol

            exs.append(jnp.exp(alpha * _INV_SQRT_C))
        return exs

    def _msg_all(b, exs):
        for grp in range(_CHUNK // 16):
            rows = rows0 + (grp * 16)
            aoff = b * _CHUNK + grp * 16
            ex = exs[grp]

            @pl.loop(0, _HC, unroll=8)
            def _msg_v_cols(c):
                cv = jnp.full((16,), c, jnp.int32)
                vcol = plsc.load_gather(kv_v[b], [rows, cv + _C])
                plsc.store_scatter(msg_v[b], [rows, cv], ex * vcol)

            @pl.loop(0, _AH, unroll=8)
            def _msg_a_cols(c):
                acol = a_v[c + cid * _AH, pl.ds(aoff, 16)]
                cv = jnp.full((16,), c, jnp.int32)
                plsc.store_scatter(msg_v[b], [rows, cv + _HC], ex * acol)

            plsc.store_scatter(msg_v[b], [rows, den_col], ex)

    def _fire_qq(b):
        return pltpu.async_copy(
            qq_hbm.at[ei_v.at[1, pl.ds(b * _CHUNK, _CHUNK)]], qq_v, sem_q)

    def _fire_kv(b):
        return pltpu.async_copy(
            kvh_hbm.at[cid].at[ei_v.at[0, pl.ds(b * _CHUNK, _CHUNK)]],
            kv_v[b], gsem[b])

    def _stash_and_scatter(b, exs):
        @pl.loop(0, _CHUNK // 16)
        def _cpd(j):
            da_v[b][0, pl.ds(j * 16, 16)] = ei_v[1, pl.ds(b * _CHUNK +
                                                          j * 16, 16)]
        _msg_all(b, exs)
        pltpu.async_copy(msg_v[b], acc_sh.at[da_v[b].at[0]], ssem[b],
                         add=True)

    @pl.loop(0, _NCHUNK, step=2)
    def _iter(g):
        ebase = (crow0 + g) * _CHUNK
        pltpu.sync_copy(ei_hbm.at[:, pl.ds(ebase, 2 * _CHUNK)], ei_v)
        pltpu.sync_copy(ea_hbm.at[:, pl.ds(ebase, 2 * _CHUNK)], a_v)
        kva = _fire_kv(0)
        kvb = _fire_kv(1)
        qqa = _fire_qq(0)

        qqa.wait()
        kva.wait()
        exs = _alpha_all(0)
        qqb = _fire_qq(1)

        @pl.when(g > 0)
        def _():
            pltpu.make_async_copy(msg_v[0], acc_sh.at[da_v[0].at[0]],
                                  ssem[0]).wait()
        _stash_and_scatter(0, exs)

        qqb.wait()
        kvb.wait()
        exs = _alpha_all(1)

        @pl.when(g > 0)
        def _():
            pltpu.make_async_copy(msg_v[1], acc_sh.at[da_v[1].at[0]],
                                  ssem[1]).wait()
        _stash_and_scatter(1, exs)

    for b in range(2):
        pltpu.make_async_copy(msg_v[b], acc_sh.at[da_v[b].at[0]],
                              ssem[b]).wait()

    plsc.subcore_barrier()

    @pl.loop(0, _RPT // _RCHUNK)
    def _wout(t):
        r0 = sid * _RPT + t * _RCHUNK
        pltpu.sync_copy(acc_sh.at[pl.ds(r0, _RCHUNK), :], zbuf)
        pltpu.sync_copy(zbuf, out_hbm.at[cid, pl.ds(r0, _RCHUNK), :])


_sc_mesh = plsc.VectorSubcoreMesh(core_axis_name="c", subcore_axis_name="s",
                                  num_cores=_NC, num_subcores=_NS)

_sc_edge = pl.kernel(
    _sc_edge_body,
    out_type=jax.ShapeDtypeStruct((_NC, _NP, _ACCW), jnp.float32),
    mesh=_sc_mesh,
    compiler_params=pltpu.CompilerParams(use_tc_tiling_on_sc=False,
                                         needs_layout_passes=False),
    scratch_types=[
        pltpu.VMEM((2, 2 * _CHUNK), jnp.int32),
        [pltpu.VMEM((1, _CHUNK), jnp.int32) for _ in range(2)],
        pltpu.VMEM((_CHUNK, _QQW), jnp.float32),
        [pltpu.VMEM((_CHUNK, _KVW), jnp.float32) for _ in range(2)],
        pltpu.VMEM((_ED, 2 * _CHUNK), jnp.float32),
        [pltpu.VMEM((_CHUNK, _ACCW), jnp.float32) for _ in range(2)],
        pltpu.VMEM((_RCHUNK, _ACCW), jnp.float32),
        pltpu.VMEM_SHARED((_NP, _ACCW), jnp.float32),
        pltpu.SemaphoreType.DMA,
        pltpu.SemaphoreType.DMA,
        pltpu.SemaphoreType.DMA,
        pltpu.SemaphoreType.DMA,
        pltpu.SemaphoreType.DMA,
    ],
)


@jax.jit
def kernel(x, edge_index, edge_attr,
           Wq0, bq0, Wk0, bk0, Wv0, bv0, We0, Ws0, bs0,
           Wq1, bq1, Wk1, bk1, Wv1, bv1, We1, Ws1, bs1,
           Wlin, blin):
    w0 = jnp.concatenate([Wq0, Wk0, Wv0, Ws0], axis=0)
    b0 = jnp.concatenate([bq0, bk0, bv0, bs0]).reshape(1, -1)
    qq0, kvh0, s0 = _proj(x, w0, b0, We0)
    eat = edge_attr.T
    parts0 = _sc_edge(qq0, kvh0, edge_index, eat)
    h1 = _post1(parts0[0], parts0[1], s0, We0)

    x2 = jnp.concatenate([h1, x], axis=1)
    w1 = jnp.concatenate([Wq1, Wk1, Wv1, Ws1], axis=0)
    b1 = jnp.concatenate([bq1, bk1, bv1, bs1]).reshape(1, -1)
    qq1, kvh1, s1 = _proj(x2, w1, b1, We1)
    parts1 = _sc_edge(qq1, kvh1, edge_index, eat)
    return _post2(parts1[0], parts1[1], s1, We1, Wlin,
                  blin.reshape(1, -1))


# speedup vs baseline: 2.4445x; 1.2004x over previous
import jax
import jax.numpy as jnp
from jax import lax
from jax.experimental import pallas as pl
from jax.experimental.pallas import tpu as pltpu
from jax.experimental.pallas import tpu_sc as plsc

_N = 10000
_E = 320000
_C = 128
_ED = 16
_HC = 64
_NC = 2
_NS = 16
_QQW = _C + _ED
_KVW = _C + _HC
_AH = _ED // 2
_ACCW = 80
_CHUNK = 80
_EPT = _E // _NS
_NCHUNK = _EPT // _CHUNK
_NP = 10240
_RPT = _NP // _NS
_RCHUNK = 32
_INV_SQRT_C = 1.0 / float(_C) ** 0.5


def _proj_body(x_ref, w_ref, b_ref, we_ref, qq_ref, kvh_ref, s_ref):
    z = lax.dot_general(x_ref[...], w_ref[...], (((1,), (1,)), ((), ())),
                        preferred_element_type=jnp.float32)
    z = z + b_ref[...]
    q = z[:, :_C]
    qw = jnp.dot(q, we_ref[...], preferred_element_type=jnp.float32)
    qq_ref[...] = jnp.concatenate([q, qw], axis=1)
    k = z[:, _C:2 * _C]
    v = z[:, 2 * _C:3 * _C]
    kvh_ref[0] = jnp.concatenate([k, v[:, :_HC]], axis=1)
    kvh_ref[1] = jnp.concatenate([k, v[:, _HC:]], axis=1)
    s_ref[...] = z[:, 3 * _C:]


def _proj(x2, wcat_t, bcat, we):
    ind = x2.shape[1]
    blk = 2000
    return pl.pallas_call(
        _proj_body,
        grid=(_N // blk,),
        in_specs=[
            pl.BlockSpec((blk, ind), lambda i: (i, 0)),
            pl.BlockSpec((4 * _C, ind), lambda i: (0, 0)),
            pl.BlockSpec((1, 4 * _C), lambda i: (0, 0)),
            pl.BlockSpec((_C, _ED), lambda i: (0, 0)),
        ],
        out_specs=[
            pl.BlockSpec((blk, _QQW), lambda i: (i, 0)),
            pl.BlockSpec((_NC, blk, _KVW), lambda i: (0, i, 0)),
            pl.BlockSpec((blk, _C), lambda i: (i, 0)),
        ],
        out_shape=[
            jax.ShapeDtypeStruct((_N, _QQW), jnp.float32),
            jax.ShapeDtypeStruct((_NC, _N, _KVW), jnp.float32),
            jax.ShapeDtypeStruct((_N, _C), jnp.float32),
        ],
    )(x2, wcat_t, bcat, we)


def _post_body(a0_ref, a1_ref, s_ref, wet_ref, h_ref):
    num = jnp.concatenate([a0_ref[:, :_HC], a1_ref[:, :_HC]], axis=1)
    t = jnp.concatenate([a0_ref[:, _HC:_HC + _AH],
                         a1_ref[:, _HC:_HC + _AH]], axis=1)
    num = num + lax.dot_general(t, wet_ref[...], (((1,), (1,)), ((), ())),
                                preferred_element_type=jnp.float32)
    den = a0_ref[:, _HC + _AH:_HC + _AH + 1]
    h_ref[...] = num / (den + 1e-16) + s_ref[...]


def _post1(a0, a1, s, wet):
    blk = 2000
    return pl.pallas_call(
        _post_body,
        grid=(_N // blk,),
        in_specs=[
            pl.BlockSpec((blk, _ACCW), lambda i: (i, 0)),
            pl.BlockSpec((blk, _ACCW), lambda i: (i, 0)),
            pl.BlockSpec((blk, _C), lambda i: (i, 0)),
            pl.BlockSpec((_C, _ED), lambda i: (0, 0)),
        ],
        out_specs=pl.BlockSpec((blk, _C), lambda i: (i, 0)),
        out_shape=jax.ShapeDtypeStruct((_N, _C), jnp.float32),
    )(a0, a1, s, wet)


def _post2_body(a0_ref, a1_ref, s_ref, wet_ref, wlt_ref, bl_ref, o_ref):
    num = jnp.concatenate([a0_ref[:, :_HC], a1_ref[:, :_HC]], axis=1)
    t = jnp.concatenate([a0_ref[:, _HC:_HC + _AH],
                         a1_ref[:, _HC:_HC + _AH]], axis=1)
    num = num + lax.dot_general(t, wet_ref[...], (((1,), (1,)), ((), ())),
                                preferred_element_type=jnp.float32)
    den = a0_ref[:, _HC + _AH:_HC + _AH + 1]
    h = num / (den + 1e-16) + s_ref[...]
    o_ref[...] = lax.dot_general(h, wlt_ref[...], (((1,), (1,)), ((), ())),
                                 preferred_element_type=jnp.float32)
    o_ref[...] += bl_ref[...]


def _post2(a0, a1, s, wet, wlt, bl):
    blk = 2000
    return pl.pallas_call(
        _post2_body,
        grid=(_N // blk,),
        in_specs=[
            pl.BlockSpec((blk, _ACCW), lambda i: (i, 0)),
            pl.BlockSpec((blk, _ACCW), lambda i: (i, 0)),
            pl.BlockSpec((blk, _C), lambda i: (i, 0)),
            pl.BlockSpec((_C, _ED), lambda i: (0, 0)),
            pl.BlockSpec((_C, _C), lambda i: (0, 0)),
            pl.BlockSpec((1, _C), lambda i: (0, 0)),
        ],
        out_specs=pl.BlockSpec((blk, _C), lambda i: (i, 0)),
        out_shape=jax.ShapeDtypeStruct((_N, _C), jnp.float32),
    )(a0, a1, s, wet, wlt, bl)


def _sc_edge_body(qq_hbm, kvh_hbm, ei_hbm, ea_hbm, out_hbm,
                  ei_v, da_v, qq_v, kv_v, a_v, msg_v, zbuf, acc_sh,
                  sem_q, sem_g0, sem_g1, sem_s0, sem_s1):
    cid = lax.axis_index("c")
    sid = lax.axis_index("s")
    zero16 = jnp.zeros((16,), jnp.float32)

    @pl.loop(0, _RCHUNK)
    def _zr(r):
        @pl.loop(0, _ACCW // 16)
        def _zc(c):
            zbuf[r, pl.ds(c * 16, 16)] = zero16

    for b in range(2):
        @pl.loop(0, _CHUNK)
        def _zm(r):
            msg_v[b][r, pl.ds(_HC, 16)] = zero16

    @pl.loop(0, _RPT // _RCHUNK)
    def _zi(t):
        pltpu.sync_copy(zbuf, acc_sh.at[pl.ds(sid * _RPT + t * _RCHUNK,
                                              _RCHUNK), :])

    plsc.subcore_barrier()

    crow0 = sid * _NCHUNK
    rows0 = lax.iota(jnp.int32, 16)
    den_col = jnp.full((16,), _HC + _AH, jnp.int32)
    gsem = [sem_g0, sem_g1]
    ssem = [sem_s0, sem_s1]

    def _alpha_all(b):
        exs = []
        for grp in range(_CHUNK // 16):
            rows = rows0 + (grp * 16)
            aoff = b * _CHUNK + grp * 16

            @plsc.parallel_loop(0, _C, unroll=8, carry=zero16)
            def alpha(c, acc):
                cv = jnp.full((16,), c, jnp.int32)
                qcol = plsc.load_gather(qq_v, [rows, cv])
                kcol = plsc.load_gather(kv_v[b], [rows, cv])
                return acc + qcol * kcol

            @plsc.parallel_loop(0, _ED, unroll=8, carry=alpha)
            def alpha(c, acc):
                cv = jnp.full((16,), c, jnp.int32)
                qwcol = plsc.load_gather(qq_v, [rows, cv + _C])
                acol = a_v[c, pl.ds(aoff, 16)]
                return acc + qwcol * acol

            exs.append(jnp.exp(alpha * _INV_SQRT_C))
        return exs

    def _msg_all(b, exs):
        for grp in range(_CHUNK // 16):
            rows = rows0 + (grp * 16)
            aoff = b * _CHUNK + grp * 16
            ex = exs[grp]

            @plsc.parallel_loop(0, _HC, unroll=8)
            def _msg_v_cols(c):
                cv = jnp.full((16,), c, jnp.int32)
                vcol = plsc.load_gather(kv_v[b], [rows, cv + _C])
                plsc.store_scatter(msg_v[b], [rows, cv], ex * vcol)

            @plsc.parallel_loop(0, _AH, unroll=8)
            def _msg_a_cols(c):
                acol = a_v[c + cid * _AH, pl.ds(aoff, 16)]
                cv = jnp.full((16,), c, jnp.int32)
                plsc.store_scatter(msg_v[b], [rows, cv + _HC], ex * acol)

            plsc.store_scatter(msg_v[b], [rows, den_col], ex)

    def _fire_qq(b):
        return pltpu.async_copy(
            qq_hbm.at[ei_v.at[1, pl.ds(b * _CHUNK, _CHUNK)]], qq_v, sem_q)

    def _fire_kv(b):
        return pltpu.async_copy(
            kvh_hbm.at[cid].at[ei_v.at[0, pl.ds(b * _CHUNK, _CHUNK)]],
            kv_v[b], gsem[b])

    def _stash_and_scatter(b, exs):
        @pl.loop(0, _CHUNK // 16)
        def _cpd(j):
            da_v[b][0, pl.ds(j * 16, 16)] = ei_v[1, pl.ds(b * _CHUNK +
                                                          j * 16, 16)]
        _msg_all(b, exs)
        pltpu.async_copy(msg_v[b], acc_sh.at[da_v[b].at[0]], ssem[b],
                         add=True)

    @pl.loop(0, _NCHUNK, step=2)
    def _iter(g):
        ebase = (crow0 + g) * _CHUNK
        pltpu.sync_copy(ei_hbm.at[:, pl.ds(ebase, 2 * _CHUNK)], ei_v)
        pltpu.sync_copy(ea_hbm.at[:, pl.ds(ebase, 2 * _CHUNK)], a_v)
        kva = _fire_kv(0)
        kvb = _fire_kv(1)
        qqa = _fire_qq(0)

        qqa.wait()
        kva.wait()
        exs = _alpha_all(0)
        qqb = _fire_qq(1)

        @pl.when(g > 0)
        def _():
            pltpu.make_async_copy(msg_v[0], acc_sh.at[da_v[0].at[0]],
                                  ssem[0]).wait()
        _stash_and_scatter(0, exs)

        qqb.wait()
        kvb.wait()
        exs = _alpha_all(1)

        @pl.when(g > 0)
        def _():
            pltpu.make_async_copy(msg_v[1], acc_sh.at[da_v[1].at[0]],
                                  ssem[1]).wait()
        _stash_and_scatter(1, exs)

    for b in range(2):
        pltpu.make_async_copy(msg_v[b], acc_sh.at[da_v[b].at[0]],
                              ssem[b]).wait()

    plsc.subcore_barrier()

    @pl.loop(0, _RPT // _RCHUNK)
    def _wout(t):
        r0 = sid * _RPT + t * _RCHUNK
        pltpu.sync_copy(acc_sh.at[pl.ds(r0, _RCHUNK), :], zbuf)
        pltpu.sync_copy(zbuf, out_hbm.at[cid, pl.ds(r0, _RCHUNK), :])


_sc_mesh = plsc.VectorSubcoreMesh(core_axis_name="c", subcore_axis_name="s",
                                  num_cores=_NC, num_subcores=_NS)

_sc_edge = pl.kernel(
    _sc_edge_body,
    out_type=jax.ShapeDtypeStruct((_NC, _NP, _ACCW), jnp.float32),
    mesh=_sc_mesh,
    compiler_params=pltpu.CompilerParams(use_tc_tiling_on_sc=False,
                                         needs_layout_passes=False),
    scratch_types=[
        pltpu.VMEM((2, 2 * _CHUNK), jnp.int32),
        [pltpu.VMEM((1, _CHUNK), jnp.int32) for _ in range(2)],
        pltpu.VMEM((_CHUNK, _QQW), jnp.float32),
        [pltpu.VMEM((_CHUNK, _KVW), jnp.float32) for _ in range(2)],
        pltpu.VMEM((_ED, 2 * _CHUNK), jnp.float32),
        [pltpu.VMEM((_CHUNK, _ACCW), jnp.float32) for _ in range(2)],
        pltpu.VMEM((_RCHUNK, _ACCW), jnp.float32),
        pltpu.VMEM_SHARED((_NP, _ACCW), jnp.float32),
        pltpu.SemaphoreType.DMA,
        pltpu.SemaphoreType.DMA,
        pltpu.SemaphoreType.DMA,
        pltpu.SemaphoreType.DMA,
        pltpu.SemaphoreType.DMA,
    ],
)


@jax.jit
def kernel(x, edge_index, edge_attr,
           Wq0, bq0, Wk0, bk0, Wv0, bv0, We0, Ws0, bs0,
           Wq1, bq1, Wk1, bk1, Wv1, bv1, We1, Ws1, bs1,
           Wlin, blin):
    w0 = jnp.concatenate([Wq0, Wk0, Wv0, Ws0], axis=0)
    b0 = jnp.concatenate([bq0, bk0, bv0, bs0]).reshape(1, -1)
    qq0, kvh0, s0 = _proj(x, w0, b0, We0)
    eat = edge_attr.T
    parts0 = _sc_edge(qq0, kvh0, edge_index, eat)
    h1 = _post1(parts0[0], parts0[1], s0, We0)

    x2 = jnp.concatenate([h1, x], axis=1)
    w1 = jnp.concatenate([Wq1, Wk1, Wv1, Ws1], axis=0)
    b1 = jnp.concatenate([bq1, bk1, bv1, bs1]).reshape(1, -1)
    qq1, kvh1, s1 = _proj(x2, w1, b1, We1)
    parts1 = _sc_edge(qq1, kvh1, edge_index, eat)
    return _post2(parts1[0], parts1[1], s1, We1, Wlin,
                  blin.reshape(1, -1))


# speedup vs baseline: 2.4534x; 1.0037x over previous
import jax
import jax.numpy as jnp
from jax import lax
from jax.experimental import pallas as pl
from jax.experimental.pallas import tpu as pltpu
from jax.experimental.pallas import tpu_sc as plsc

_N = 10000
_E = 320000
_C = 128
_ED = 16
_HC = 64
_NC = 2
_NS = 16
_QQW = _C + _ED
_KVW = _C + _HC
_AH = _ED // 2
_ACCW = 80
_CHUNK = 80
_EPT = _E // _NS
_NCHUNK = _EPT // _CHUNK
_NP = 10240
_RPT = _NP // _NS
_RCHUNK = 32
_INV_SQRT_C = 1.0 / float(_C) ** 0.5


def _proj_body(x_ref, w_ref, b_ref, we_ref, qq_ref, kvh_ref, s_ref):
    z = lax.dot_general(x_ref[...], w_ref[...], (((1,), (1,)), ((), ())),
                        preferred_element_type=jnp.float32)
    z = z + b_ref[...]
    q = z[:, :_C]
    qw = jnp.dot(q, we_ref[...], preferred_element_type=jnp.float32)
    qq_ref[...] = jnp.concatenate([q, qw], axis=1)
    k = z[:, _C:2 * _C]
    v = z[:, 2 * _C:3 * _C]
    kvh_ref[0] = jnp.concatenate([k, v[:, :_HC]], axis=1)
    kvh_ref[1] = jnp.concatenate([k, v[:, _HC:]], axis=1)
    s_ref[...] = z[:, 3 * _C:]


def _proj(x2, wcat_t, bcat, we):
    ind = x2.shape[1]
    blk = 2000
    return pl.pallas_call(
        _proj_body,
        grid=(_N // blk,),
        in_specs=[
            pl.BlockSpec((blk, ind), lambda i: (i, 0)),
            pl.BlockSpec((4 * _C, ind), lambda i: (0, 0)),
            pl.BlockSpec((1, 4 * _C), lambda i: (0, 0)),
            pl.BlockSpec((_C, _ED), lambda i: (0, 0)),
        ],
        out_specs=[
            pl.BlockSpec((blk, _QQW), lambda i: (i, 0)),
            pl.BlockSpec((_NC, blk, _KVW), lambda i: (0, i, 0)),
            pl.BlockSpec((blk, _C), lambda i: (i, 0)),
        ],
        out_shape=[
            jax.ShapeDtypeStruct((_N, _QQW), jnp.float32),
            jax.ShapeDtypeStruct((_NC, _N, _KVW), jnp.float32),
            jax.ShapeDtypeStruct((_N, _C), jnp.float32),
        ],
    )(x2, wcat_t, bcat, we)


def _post_body(a0_ref, a1_ref, s_ref, wet_ref, h_ref):
    num = jnp.concatenate([a0_ref[:, :_HC], a1_ref[:, :_HC]], axis=1)
    t = jnp.concatenate([a0_ref[:, _HC:_HC + _AH],
                         a1_ref[:, _HC:_HC + _AH]], axis=1)
    num = num + lax.dot_general(t, wet_ref[...], (((1,), (1,)), ((), ())),
                                preferred_element_type=jnp.float32)
    den = a0_ref[:, _HC + _AH:_HC + _AH + 1]
    h_ref[...] = num / (den + 1e-16) + s_ref[...]


def _post1(a0, a1, s, wet):
    blk = 2000
    return pl.pallas_call(
        _post_body,
        grid=(_N // blk,),
        in_specs=[
            pl.BlockSpec((blk, _ACCW), lambda i: (i, 0)),
            pl.BlockSpec((blk, _ACCW), lambda i: (i, 0)),
            pl.BlockSpec((blk, _C), lambda i: (i, 0)),
            pl.BlockSpec((_C, _ED), lambda i: (0, 0)),
        ],
        out_specs=pl.BlockSpec((blk, _C), lambda i: (i, 0)),
        out_shape=jax.ShapeDtypeStruct((_N, _C), jnp.float32),
    )(a0, a1, s, wet)


def _post2_body(a0_ref, a1_ref, s_ref, wet_ref, wlt_ref, bl_ref, o_ref):
    num = jnp.concatenate([a0_ref[:, :_HC], a1_ref[:, :_HC]], axis=1)
    t = jnp.concatenate([a0_ref[:, _HC:_HC + _AH],
                         a1_ref[:, _HC:_HC + _AH]], axis=1)
    num = num + lax.dot_general(t, wet_ref[...], (((1,), (1,)), ((), ())),
                                preferred_element_type=jnp.float32)
    den = a0_ref[:, _HC + _AH:_HC + _AH + 1]
    h = num / (den + 1e-16) + s_ref[...]
    o_ref[...] = lax.dot_general(h, wlt_ref[...], (((1,), (1,)), ((), ())),
                                 preferred_element_type=jnp.float32)
    o_ref[...] += bl_ref[...]


def _post2(a0, a1, s, wet, wlt, bl):
    blk = 2000
    return pl.pallas_call(
        _post2_body,
        grid=(_N // blk,),
        in_specs=[
            pl.BlockSpec((blk, _ACCW), lambda i: (i, 0)),
            pl.BlockSpec((blk, _ACCW), lambda i: (i, 0)),
            pl.BlockSpec((blk, _C), lambda i: (i, 0)),
            pl.BlockSpec((_C, _ED), lambda i: (0, 0)),
            pl.BlockSpec((_C, _C), lambda i: (0, 0)),
            pl.BlockSpec((1, _C), lambda i: (0, 0)),
        ],
        out_specs=pl.BlockSpec((blk, _C), lambda i: (i, 0)),
        out_shape=jax.ShapeDtypeStruct((_N, _C), jnp.float32),
    )(a0, a1, s, wet, wlt, bl)


def _sc_edge_body(qq_hbm, kvh_hbm, ei_hbm, ea_hbm, out_hbm,
                  ei_v, da_v, qq_v, kv_v, a_v, msg_v, zbuf, acc_sh,
                  sem_q, sem_g0, sem_g1, sem_s0, sem_s1, sem_i):
    cid = lax.axis_index("c")
    sid = lax.axis_index("s")
    zero16 = jnp.zeros((16,), jnp.float32)

    @pl.loop(0, _RCHUNK)
    def _zr(r):
        @pl.loop(0, _ACCW // 16)
        def _zc(c):
            zbuf[r, pl.ds(c * 16, 16)] = zero16

    for b in range(2):
        @pl.loop(0, _CHUNK)
        def _zm(r):
            msg_v[b][r, pl.ds(_HC, 16)] = zero16

    @pl.loop(0, _RPT // _RCHUNK)
    def _zi(t):
        pltpu.sync_copy(zbuf, acc_sh.at[pl.ds(sid * _RPT + t * _RCHUNK,
                                              _RCHUNK), :])

    plsc.subcore_barrier()

    crow0 = sid * _NCHUNK
    rows0 = lax.iota(jnp.int32, 16)
    den_col = jnp.full((16,), _HC + _AH, jnp.int32)
    gsem = [sem_g0, sem_g1]
    ssem = [sem_s0, sem_s1]

    def _alpha_all(b):
        exs = []
        for grp in range(_CHUNK // 16):
            rows = rows0 + (grp * 16)
            aoff = b * _CHUNK + grp * 16

            @plsc.parallel_loop(0, _C, unroll=8, carry=zero16)
            def alpha(c, acc):
                cv = jnp.full((16,), c, jnp.int32)
                qcol = plsc.load_gather(qq_v, [rows, cv])
                kcol = plsc.load_gather(kv_v[b], [rows, cv])
                return acc + qcol * kcol

            @plsc.parallel_loop(0, _ED, unroll=8, carry=alpha)
            def alpha(c, acc):
                cv = jnp.full((16,), c, jnp.int32)
                qwcol = plsc.load_gather(qq_v, [rows, cv + _C])
                acol = a_v[c, pl.ds(aoff, 16)]
                return acc + qwcol * acol

            exs.append(jnp.exp(alpha * _INV_SQRT_C))
        return exs

    def _msg_all(b, exs):
        for grp in range(_CHUNK // 16):
            rows = rows0 + (grp * 16)
            aoff = b * _CHUNK + grp * 16
            ex = exs[grp]

            @plsc.parallel_loop(0, _HC, unroll=8)
            def _msg_v_cols(c):
                cv = jnp.full((16,), c, jnp.int32)
                vcol = plsc.load_gather(kv_v[b], [rows, cv + _C])
                plsc.store_scatter(msg_v[b], [rows, cv], ex * vcol)

            @plsc.parallel_loop(0, _AH, unroll=8)
            def _msg_a_cols(c):
                acol = a_v[c + cid * _AH, pl.ds(aoff, 16)]
                cv = jnp.full((16,), c, jnp.int32)
                plsc.store_scatter(msg_v[b], [rows, cv + _HC], ex * acol)

            plsc.store_scatter(msg_v[b], [rows, den_col], ex)

    def _fire_qq(b):
        return pltpu.async_copy(
            qq_hbm.at[ei_v.at[1, pl.ds(b * _CHUNK, _CHUNK)]], qq_v, sem_q)

    def _fire_kv(b):
        return pltpu.async_copy(
            kvh_hbm.at[cid].at[ei_v.at[0, pl.ds(b * _CHUNK, _CHUNK)]],
            kv_v[b], gsem[b])

    def _fetch_idx(g):
        c1 = pltpu.async_copy(
            ei_hbm.at[:, pl.ds((crow0 + g) * _CHUNK, 2 * _CHUNK)], ei_v,
            sem_i)
        c2 = pltpu.async_copy(
            ea_hbm.at[:, pl.ds((crow0 + g) * _CHUNK, 2 * _CHUNK)], a_v,
            sem_i)
        return c1, c2

    def _stash_and_scatter(b, exs):
        @pl.loop(0, _CHUNK // 16)
        def _cpd(j):
            da_v[b][0, pl.ds(j * 16, 16)] = ei_v[1, pl.ds(b * _CHUNK +
                                                          j * 16, 16)]
        _msg_all(b, exs)
        pltpu.async_copy(msg_v[b], acc_sh.at[da_v[b].at[0]], ssem[b],
                         add=True)

    _fetch_idx(0)

    @pl.loop(0, _NCHUNK, step=2)
    def _iter(g):
        ebase = (crow0 + g) * _CHUNK
        pltpu.make_async_copy(ei_hbm.at[:, pl.ds(ebase, 2 * _CHUNK)], ei_v,
                              sem_i).wait()
        pltpu.make_async_copy(ea_hbm.at[:, pl.ds(ebase, 2 * _CHUNK)], a_v,
                              sem_i).wait()
        kva = _fire_kv(0)
        kvb = _fire_kv(1)
        qqa = _fire_qq(0)

        qqa.wait()
        kva.wait()
        exs = _alpha_all(0)
        qqb = _fire_qq(1)

        @pl.when(g > 0)
        def _():
            pltpu.make_async_copy(msg_v[0], acc_sh.at[da_v[0].at[0]],
                                  ssem[0]).wait()
        _stash_and_scatter(0, exs)

        qqb.wait()
        kvb.wait()
        exs = _alpha_all(1)

        @pl.when(g > 0)
        def _():
            pltpu.make_async_copy(msg_v[1], acc_sh.at[da_v[1].at[0]],
                                  ssem[1]).wait()
        _stash_and_scatter(1, exs)

        @pl.when(g + 2 < _NCHUNK)
        def _():
            _fetch_idx(g + 2)

    for b in range(2):
        pltpu.make_async_copy(msg_v[b], acc_sh.at[da_v[b].at[0]],
                              ssem[b]).wait()

    plsc.subcore_barrier()

    @pl.loop(0, _RPT // _RCHUNK)
    def _wout(t):
        r0 = sid * _RPT + t * _RCHUNK
        pltpu.sync_copy(acc_sh.at[pl.ds(r0, _RCHUNK), :], zbuf)
        pltpu.sync_copy(zbuf, out_hbm.at[cid, pl.ds(r0, _RCHUNK), :])


_sc_mesh = plsc.VectorSubcoreMesh(core_axis_name="c", subcore_axis_name="s",
                                  num_cores=_NC, num_subcores=_NS)

_sc_edge = pl.kernel(
    _sc_edge_body,
    out_type=jax.ShapeDtypeStruct((_NC, _NP, _ACCW), jnp.float32),
    mesh=_sc_mesh,
    compiler_params=pltpu.CompilerParams(use_tc_tiling_on_sc=False,
                                         needs_layout_passes=False),
    scratch_types=[
        pltpu.VMEM((2, 2 * _CHUNK), jnp.int32),
        [pltpu.VMEM((1, _CHUNK), jnp.int32) for _ in range(2)],
        pltpu.VMEM((_CHUNK, _QQW), jnp.float32),
        [pltpu.VMEM((_CHUNK, _KVW), jnp.float32) for _ in range(2)],
        pltpu.VMEM((_ED, 2 * _CHUNK), jnp.float32),
        [pltpu.VMEM((_CHUNK, _ACCW), jnp.float32) for _ in range(2)],
        pltpu.VMEM((_RCHUNK, _ACCW), jnp.float32),
        pltpu.VMEM_SHARED((_NP, _ACCW), jnp.float32),
        pltpu.SemaphoreType.DMA,
        pltpu.SemaphoreType.DMA,
        pltpu.SemaphoreType.DMA,
        pltpu.SemaphoreType.DMA,
        pltpu.SemaphoreType.DMA,
        pltpu.SemaphoreType.DMA,
    ],
)


@jax.jit
def kernel(x, edge_index, edge_attr,
           Wq0, bq0, Wk0, bk0, Wv0, bv0, We0, Ws0, bs0,
           Wq1, bq1, Wk1, bk1, Wv1, bv1, We1, Ws1, bs1,
           Wlin, blin):
    w0 = jnp.concatenate([Wq0, Wk0, Wv0, Ws0], axis=0)
    b0 = jnp.concatenate([bq0, bk0, bv0, bs0]).reshape(1, -1)
    qq0, kvh0, s0 = _proj(x, w0, b0, We0)
    eat = edge_attr.T
    parts0 = _sc_edge(qq0, kvh0, edge_index, eat)
    h1 = _post1(parts0[0], parts0[1], s0, We0)

    x2 = jnp.concatenate([h1, x], axis=1)
    w1 = jnp.concatenate([Wq1, Wk1, Wv1, Ws1], axis=0)
    b1 = jnp.concatenate([bq1, bk1, bv1, bs1]).reshape(1, -1)
    qq1, kvh1, s1 = _proj(x2, w1, b1, We1)
    parts1 = _sc_edge(qq1, kvh1, edge_index, eat)
    return _post2(parts1[0], parts1[1], s1, We1, Wlin,
                  blin.reshape(1, -1))


# speedup vs baseline: 6.9982x; 2.8524x over previous
import jax
import jax.numpy as jnp
from jax import lax
from jax.experimental import pallas as pl
from jax.experimental.pallas import tpu as pltpu
from jax.experimental.pallas import tpu_sc as plsc

_N = 10000
_E = 320000
_C = 128
_ED = 16
_HC = 64
_NC = 2
_NS = 16
_QQW = _C + _ED
_KVW = _C + _HC
_AH = _ED // 2
_ACCW = 80
_CHUNK = 80
_EPT = _E // _NS
_NCHUNK = _EPT // _CHUNK
_NP = 10240
_RPT = _NP // _NS
_RCHUNK = 32
_INV_SQRT_C = 1.0 / float(_C) ** 0.5


def _proj_body(x_ref, w_ref, b_ref, we_ref, qq_ref, kvh_ref, s_ref):
    z = lax.dot_general(x_ref[...], w_ref[...], (((1,), (1,)), ((), ())),
                        preferred_element_type=jnp.float32)
    z = z + b_ref[...]
    q = z[:, :_C]
    qw = jnp.dot(q, we_ref[...], preferred_element_type=jnp.float32)
    qq_ref[...] = jnp.concatenate([q, qw], axis=1)
    k = z[:, _C:2 * _C]
    v = z[:, 2 * _C:3 * _C]
    kvh_ref[0] = jnp.concatenate([k, v[:, :_HC]], axis=1)
    kvh_ref[1] = jnp.concatenate([k, v[:, _HC:]], axis=1)
    s_ref[...] = z[:, 3 * _C:]


def _proj(x2, wcat_t, bcat, we):
    ind = x2.shape[1]
    blk = 2000
    return pl.pallas_call(
        _proj_body,
        grid=(_N // blk,),
        in_specs=[
            pl.BlockSpec((blk, ind), lambda i: (i, 0)),
            pl.BlockSpec((4 * _C, ind), lambda i: (0, 0)),
            pl.BlockSpec((1, 4 * _C), lambda i: (0, 0)),
            pl.BlockSpec((_C, _ED), lambda i: (0, 0)),
        ],
        out_specs=[
            pl.BlockSpec((blk, _QQW), lambda i: (i, 0)),
            pl.BlockSpec((_NC, blk, _KVW), lambda i: (0, i, 0)),
            pl.BlockSpec((blk, _C), lambda i: (i, 0)),
        ],
        out_shape=[
            jax.ShapeDtypeStruct((_N, _QQW), jnp.float32),
            jax.ShapeDtypeStruct((_NC, _N, _KVW), jnp.float32),
            jax.ShapeDtypeStruct((_N, _C), jnp.float32),
        ],
    )(x2, wcat_t, bcat, we)


def _post_body(a0_ref, a1_ref, s_ref, wet_ref, h_ref):
    num = jnp.concatenate([a0_ref[:, :_HC], a1_ref[:, :_HC]], axis=1)
    t = jnp.concatenate([a0_ref[:, _HC:_HC + _AH],
                         a1_ref[:, _HC:_HC + _AH]], axis=1)
    num = num + lax.dot_general(t, wet_ref[...], (((1,), (1,)), ((), ())),
                                preferred_element_type=jnp.float32)
    den = a0_ref[:, _HC + _AH:_HC + _AH + 1]
    h_ref[...] = num / (den + 1e-16) + s_ref[...]


def _post1(a0, a1, s, wet):
    blk = 2000
    return pl.pallas_call(
        _post_body,
        grid=(_N // blk,),
        in_specs=[
            pl.BlockSpec((blk, _ACCW), lambda i: (i, 0)),
            pl.BlockSpec((blk, _ACCW), lambda i: (i, 0)),
            pl.BlockSpec((blk, _C), lambda i: (i, 0)),
            pl.BlockSpec((_C, _ED), lambda i: (0, 0)),
        ],
        out_specs=pl.BlockSpec((blk, _C), lambda i: (i, 0)),
        out_shape=jax.ShapeDtypeStruct((_N, _C), jnp.float32),
    )(a0, a1, s, wet)


def _post2_body(a0_ref, a1_ref, s_ref, wet_ref, wlt_ref, bl_ref, o_ref):
    num = jnp.concatenate([a0_ref[:, :_HC], a1_ref[:, :_HC]], axis=1)
    t = jnp.concatenate([a0_ref[:, _HC:_HC + _AH],
                         a1_ref[:, _HC:_HC + _AH]], axis=1)
    num = num + lax.dot_general(t, wet_ref[...], (((1,), (1,)), ((), ())),
                                preferred_element_type=jnp.float32)
    den = a0_ref[:, _HC + _AH:_HC + _AH + 1]
    h = num / (den + 1e-16) + s_ref[...]
    o_ref[...] = lax.dot_general(h, wlt_ref[...], (((1,), (1,)), ((), ())),
                                 preferred_element_type=jnp.float32)
    o_ref[...] += bl_ref[...]


def _post2(a0, a1, s, wet, wlt, bl):
    blk = 2000
    return pl.pallas_call(
        _post2_body,
        grid=(_N // blk,),
        in_specs=[
            pl.BlockSpec((blk, _ACCW), lambda i: (i, 0)),
            pl.BlockSpec((blk, _ACCW), lambda i: (i, 0)),
            pl.BlockSpec((blk, _C), lambda i: (i, 0)),
            pl.BlockSpec((_C, _ED), lambda i: (0, 0)),
            pl.BlockSpec((_C, _C), lambda i: (0, 0)),
            pl.BlockSpec((1, _C), lambda i: (0, 0)),
        ],
        out_specs=pl.BlockSpec((blk, _C), lambda i: (i, 0)),
        out_shape=jax.ShapeDtypeStruct((_N, _C), jnp.float32),
    )(a0, a1, s, wet, wlt, bl)


def _sc_edge_body(qq_hbm, kvh_hbm, ei_hbm, ea_hbm, out_hbm,
                  ei_v, da_v, qq_v, kv_v, a_v, msg_v, zbuf, acc_sh,
                  sem_q, sem_g0, sem_g1, sem_s0, sem_s1, sem_i):
    cid = lax.axis_index("c")
    sid = lax.axis_index("s")
    zero16 = jnp.zeros((16,), jnp.float32)

    @pl.loop(0, _RCHUNK)
    def _zr(r):
        @pl.loop(0, _ACCW // 16)
        def _zc(c):
            zbuf[r, pl.ds(c * 16, 16)] = zero16

    for b in range(2):
        @pl.loop(0, _CHUNK)
        def _zm(r):
            msg_v[b][r, pl.ds(_HC, 16)] = zero16

    @pl.loop(0, _RPT // _RCHUNK)
    def _zi(t):
        pltpu.sync_copy(zbuf, acc_sh.at[pl.ds(sid * _RPT + t * _RCHUNK,
                                              _RCHUNK), :])

    plsc.subcore_barrier()

    crow0 = sid * _NCHUNK
    rows0 = lax.iota(jnp.int32, 16)
    den_col = jnp.full((16,), _HC + _AH, jnp.int32)
    gsem = [sem_g0, sem_g1]
    ssem = [sem_s0, sem_s1]

    def _alpha_all(b):
        exs = []
        for grp in range(_CHUNK // 16):
            rows = rows0 + (grp * 16)
            aoff = b * _CHUNK + grp * 16

            av = jnp.full((16,), aoff, jnp.int32) + rows0

            @plsc.parallel_loop(0, _C, unroll=8, carry=zero16)
            def alpha(c, acc):
                cv = (jnp.full((16,), c, jnp.int32) + rows0) & (_C - 1)
                qcol = plsc.load_gather(qq_v, [rows, cv])
                kcol = plsc.load_gather(kv_v[b], [rows, cv])
                return acc + qcol * kcol

            @plsc.parallel_loop(0, _ED, unroll=8, carry=alpha)
            def alpha(c, acc):
                cr = (jnp.full((16,), c, jnp.int32) + rows0) & (_ED - 1)
                qwcol = plsc.load_gather(qq_v, [rows, cr + _C])
                acol = plsc.load_gather(a_v, [cr, av])
                return acc + qwcol * acol

            exs.append(jnp.exp(alpha * _INV_SQRT_C))
        return exs

    def _msg_all(b, exs):
        for grp in range(_CHUNK // 16):
            rows = rows0 + (grp * 16)
            aoff = b * _CHUNK + grp * 16
            ex = exs[grp]

            av = jnp.full((16,), aoff, jnp.int32) + rows0

            @plsc.parallel_loop(0, _HC, unroll=8)
            def _msg_v_cols(c):
                cr = (jnp.full((16,), c, jnp.int32) + rows0) & (_HC - 1)
                vcol = plsc.load_gather(kv_v[b], [rows, cr + _C])
                plsc.store_scatter(msg_v[b], [rows, cr], ex * vcol)

            @plsc.parallel_loop(0, _AH, unroll=8)
            def _msg_a_cols(c):
                cr = (jnp.full((16,), c, jnp.int32) + rows0) & (_AH - 1)
                acol = plsc.load_gather(a_v, [cr + cid * _AH, av])
                plsc.store_scatter(msg_v[b], [rows, cr + _HC], ex * acol)

            plsc.store_scatter(msg_v[b], [rows, den_col], ex)

    def _fire_qq(b):
        return pltpu.async_copy(
            qq_hbm.at[ei_v.at[1, pl.ds(b * _CHUNK, _CHUNK)]], qq_v, sem_q)

    def _fire_kv(b):
        return pltpu.async_copy(
            kvh_hbm.at[cid].at[ei_v.at[0, pl.ds(b * _CHUNK, _CHUNK)]],
            kv_v[b], gsem[b])

    def _fetch_idx(g):
        c1 = pltpu.async_copy(
            ei_hbm.at[:, pl.ds((crow0 + g) * _CHUNK, 2 * _CHUNK)], ei_v,
            sem_i)
        c2 = pltpu.async_copy(
            ea_hbm.at[:, pl.ds((crow0 + g) * _CHUNK, 2 * _CHUNK)], a_v,
            sem_i)
        return c1, c2

    def _stash_and_scatter(b, exs):
        @pl.loop(0, _CHUNK // 16)
        def _cpd(j):
            da_v[b][0, pl.ds(j * 16, 16)] = ei_v[1, pl.ds(b * _CHUNK +
                                                          j * 16, 16)]
        _msg_all(b, exs)
        pltpu.async_copy(msg_v[b], acc_sh.at[da_v[b].at[0]], ssem[b],
                         add=True)

    _fetch_idx(0)

    @pl.loop(0, _NCHUNK, step=2)
    def _iter(g):
        ebase = (crow0 + g) * _CHUNK
        pltpu.make_async_copy(ei_hbm.at[:, pl.ds(ebase, 2 * _CHUNK)], ei_v,
                              sem_i).wait()
        pltpu.make_async_copy(ea_hbm.at[:, pl.ds(ebase, 2 * _CHUNK)], a_v,
                              sem_i).wait()
        kva = _fire_kv(0)
        kvb = _fire_kv(1)
        qqa = _fire_qq(0)

        qqa.wait()
        kva.wait()
        exs = _alpha_all(0)
        qqb = _fire_qq(1)

        @pl.when(g > 0)
        def _():
            pltpu.make_async_copy(msg_v[0], acc_sh.at[da_v[0].at[0]],
                                  ssem[0]).wait()
        _stash_and_scatter(0, exs)

        qqb.wait()
        kvb.wait()
        exs = _alpha_all(1)

        @pl.when(g > 0)
        def _():
            pltpu.make_async_copy(msg_v[1], acc_sh.at[da_v[1].at[0]],
                                  ssem[1]).wait()
        _stash_and_scatter(1, exs)

        @pl.when(g + 2 < _NCHUNK)
        def _():
            _fetch_idx(g + 2)

    for b in range(2):
        pltpu.make_async_copy(msg_v[b], acc_sh.at[da_v[b].at[0]],
                              ssem[b]).wait()

    plsc.subcore_barrier()

    @pl.loop(0, _RPT // _RCHUNK)
    def _wout(t):
        r0 = sid * _RPT + t * _RCHUNK
        pltpu.sync_copy(acc_sh.at[pl.ds(r0, _RCHUNK), :], zbuf)
        pltpu.sync_copy(zbuf, out_hbm.at[cid, pl.ds(r0, _RCHUNK), :])


_sc_mesh = plsc.VectorSubcoreMesh(core_axis_name="c", subcore_axis_name="s",
                                  num_cores=_NC, num_subcores=_NS)

_sc_edge = pl.kernel(
    _sc_edge_body,
    out_type=jax.ShapeDtypeStruct((_NC, _NP, _ACCW), jnp.float32),
    mesh=_sc_mesh,
    compiler_params=pltpu.CompilerParams(use_tc_tiling_on_sc=False,
                                         needs_layout_passes=False),
    scratch_types=[
        pltpu.VMEM((2, 2 * _CHUNK), jnp.int32),
        [pltpu.VMEM((1, _CHUNK), jnp.int32) for _ in range(2)],
        pltpu.VMEM((_CHUNK, _QQW), jnp.float32),
        [pltpu.VMEM((_CHUNK, _KVW), jnp.float32) for _ in range(2)],
        pltpu.VMEM((_ED, 2 * _CHUNK), jnp.float32),
        [pltpu.VMEM((_CHUNK, _ACCW), jnp.float32) for _ in range(2)],
        pltpu.VMEM((_RCHUNK, _ACCW), jnp.float32),
        pltpu.VMEM_SHARED((_NP, _ACCW), jnp.float32),
        pltpu.SemaphoreType.DMA,
        pltpu.SemaphoreType.DMA,
        pltpu.SemaphoreType.DMA,
        pltpu.SemaphoreType.DMA,
        pltpu.SemaphoreType.DMA,
        pltpu.SemaphoreType.DMA,
    ],
)


@jax.jit
def kernel(x, edge_index, edge_attr,
           Wq0, bq0, Wk0, bk0, Wv0, bv0, We0, Ws0, bs0,
           Wq1, bq1, Wk1, bk1, Wv1, bv1, We1, Ws1, bs1,
           Wlin, blin):
    w0 = jnp.concatenate([Wq0, Wk0, Wv0, Ws0], axis=0)
    b0 = jnp.concatenate([bq0, bk0, bv0, bs0]).reshape(1, -1)
    qq0, kvh0, s0 = _proj(x, w0, b0, We0)
    eat = edge_attr.T
    parts0 = _sc_edge(qq0, kvh0, edge_index, eat)
    h1 = _post1(parts0[0], parts0[1], s0, We0)

    x2 = jnp.concatenate([h1, x], axis=1)
    w1 = jnp.concatenate([Wq1, Wk1, Wv1, Ws1], axis=0)
    b1 = jnp.concatenate([bq1, bk1, bv1, bs1]).reshape(1, -1)
    qq1, kvh1, s1 = _proj(x2, w1, b1, We1)
    parts1 = _sc_edge(qq1, kvh1, edge_index, eat)
    return _post2(parts1[0], parts1[1], s1, We1, Wlin,
                  blin.reshape(1, -1))


# speedup vs baseline: 7.3380x; 1.0486x over previous
import jax
import jax.numpy as jnp
from jax import lax
from jax.experimental import pallas as pl
from jax.experimental.pallas import tpu as pltpu
from jax.experimental.pallas import tpu_sc as plsc

_N = 10000
_E = 320000
_C = 128
_ED = 16
_HC = 64
_NC = 2
_NS = 16
_QQW = _C + _ED
_KVW = _C + _HC
_AH = _ED // 2
_ACCW = 80
_CHUNK = 80
_EPT = _E // _NS
_NCHUNK = _EPT // _CHUNK
_NP = 10240
_RPT = _NP // _NS
_RCHUNK = 32
_INV_SQRT_C = 1.0 / float(_C) ** 0.5


def _proj_body(x_ref, w_ref, b_ref, we_ref, qq_ref, kvh_ref, s_ref):
    z = lax.dot_general(x_ref[...], w_ref[...], (((1,), (1,)), ((), ())),
                        preferred_element_type=jnp.float32)
    z = z + b_ref[...]
    q = z[:, :_C]
    qw = jnp.dot(q, we_ref[...], preferred_element_type=jnp.float32)
    qq_ref[...] = jnp.concatenate([q, qw], axis=1)
    k = z[:, _C:2 * _C]
    v = z[:, 2 * _C:3 * _C]
    kvh_ref[0] = jnp.concatenate([k, v[:, :_HC]], axis=1)
    kvh_ref[1] = jnp.concatenate([k, v[:, _HC:]], axis=1)
    s_ref[...] = z[:, 3 * _C:]


def _proj(x2, wcat_t, bcat, we):
    ind = x2.shape[1]
    blk = 2000
    return pl.pallas_call(
        _proj_body,
        grid=(_N // blk,),
        in_specs=[
            pl.BlockSpec((blk, ind), lambda i: (i, 0)),
            pl.BlockSpec((4 * _C, ind), lambda i: (0, 0)),
            pl.BlockSpec((1, 4 * _C), lambda i: (0, 0)),
            pl.BlockSpec((_C, _ED), lambda i: (0, 0)),
        ],
        out_specs=[
            pl.BlockSpec((blk, _QQW), lambda i: (i, 0)),
            pl.BlockSpec((_NC, blk, _KVW), lambda i: (0, i, 0)),
            pl.BlockSpec((blk, _C), lambda i: (i, 0)),
        ],
        out_shape=[
            jax.ShapeDtypeStruct((_N, _QQW), jnp.float32),
            jax.ShapeDtypeStruct((_NC, _N, _KVW), jnp.float32),
            jax.ShapeDtypeStruct((_N, _C), jnp.float32),
        ],
    )(x2, wcat_t, bcat, we)


def _post_body(a0_ref, a1_ref, s_ref, wet_ref, h_ref):
    num = jnp.concatenate([a0_ref[:, :_HC], a1_ref[:, :_HC]], axis=1)
    t = jnp.concatenate([a0_ref[:, _HC:_HC + _AH],
                         a1_ref[:, _HC:_HC + _AH]], axis=1)
    num = num + lax.dot_general(t, wet_ref[...], (((1,), (1,)), ((), ())),
                                preferred_element_type=jnp.float32)
    den = a0_ref[:, _HC + _AH:_HC + _AH + 1]
    h_ref[...] = num / (den + 1e-16) + s_ref[...]


def _post1(a0, a1, s, wet):
    blk = 2000
    return pl.pallas_call(
        _post_body,
        grid=(_N // blk,),
        in_specs=[
            pl.BlockSpec((blk, _ACCW), lambda i: (i, 0)),
            pl.BlockSpec((blk, _ACCW), lambda i: (i, 0)),
            pl.BlockSpec((blk, _C), lambda i: (i, 0)),
            pl.BlockSpec((_C, _ED), lambda i: (0, 0)),
        ],
        out_specs=pl.BlockSpec((blk, _C), lambda i: (i, 0)),
        out_shape=jax.ShapeDtypeStruct((_N, _C), jnp.float32),
    )(a0, a1, s, wet)


def _post2_body(a0_ref, a1_ref, s_ref, wet_ref, wlt_ref, bl_ref, o_ref):
    num = jnp.concatenate([a0_ref[:, :_HC], a1_ref[:, :_HC]], axis=1)
    t = jnp.concatenate([a0_ref[:, _HC:_HC + _AH],
                         a1_ref[:, _HC:_HC + _AH]], axis=1)
    num = num + lax.dot_general(t, wet_ref[...], (((1,), (1,)), ((), ())),
                                preferred_element_type=jnp.float32)
    den = a0_ref[:, _HC + _AH:_HC + _AH + 1]
    h = num / (den + 1e-16) + s_ref[...]
    o_ref[...] = lax.dot_general(h, wlt_ref[...], (((1,), (1,)), ((), ())),
                                 preferred_element_type=jnp.float32)
    o_ref[...] += bl_ref[...]


def _post2(a0, a1, s, wet, wlt, bl):
    blk = 2000
    return pl.pallas_call(
        _post2_body,
        grid=(_N // blk,),
        in_specs=[
            pl.BlockSpec((blk, _ACCW), lambda i: (i, 0)),
            pl.BlockSpec((blk, _ACCW), lambda i: (i, 0)),
            pl.BlockSpec((blk, _C), lambda i: (i, 0)),
            pl.BlockSpec((_C, _ED), lambda i: (0, 0)),
            pl.BlockSpec((_C, _C), lambda i: (0, 0)),
            pl.BlockSpec((1, _C), lambda i: (0, 0)),
        ],
        out_specs=pl.BlockSpec((blk, _C), lambda i: (i, 0)),
        out_shape=jax.ShapeDtypeStruct((_N, _C), jnp.float32),
    )(a0, a1, s, wet, wlt, bl)


def _sc_edge_body(qq_hbm, kvh_hbm, ei_hbm, ea_hbm, out_hbm,
                  ei_v, da_v, qq_v, kv_v, a_v, msg_v, zbuf, acc_sh,
                  sem_q, sem_g0, sem_g1, sem_s0, sem_s1, sem_i):
    cid = lax.axis_index("c")
    sid = lax.axis_index("s")
    zero16 = jnp.zeros((16,), jnp.float32)

    @pl.loop(0, _RCHUNK)
    def _zr(r):
        @pl.loop(0, _ACCW // 16)
        def _zc(c):
            zbuf[r, pl.ds(c * 16, 16)] = zero16

    for b in range(2):
        @pl.loop(0, _CHUNK)
        def _zm(r):
            msg_v[b][r, pl.ds(_HC, 16)] = zero16

    @pl.loop(0, _RPT // _RCHUNK)
    def _zi(t):
        pltpu.sync_copy(zbuf, acc_sh.at[pl.ds(sid * _RPT + t * _RCHUNK,
                                              _RCHUNK), :])

    plsc.subcore_barrier()

    crow0 = sid * _NCHUNK
    rows0 = lax.iota(jnp.int32, 16)
    den_col = jnp.full((16,), _HC + _AH, jnp.int32)
    gsem = [sem_g0, sem_g1]
    ssem = [sem_s0, sem_s1]

    def _alpha_all(b):
        exs = []
        for grp in range(_CHUNK // 16):
            rows = rows0 + (grp * 16)
            aoff = b * _CHUNK + grp * 16

            av = jnp.full((16,), aoff, jnp.int32) + rows0

            @plsc.parallel_loop(0, _C, unroll=8, carry=zero16)
            def alpha(c, acc):
                cv = (jnp.full((16,), c, jnp.int32) + rows0) & (_C - 1)
                qcol = plsc.load_gather(qq_v[b], [rows, cv])
                kcol = plsc.load_gather(kv_v[b], [rows, cv])
                return acc + qcol * kcol

            @plsc.parallel_loop(0, _ED, unroll=8, carry=alpha)
            def alpha(c, acc):
                cr = (jnp.full((16,), c, jnp.int32) + rows0) & (_ED - 1)
                qwcol = plsc.load_gather(qq_v[b], [rows, cr + _C])
                acol = plsc.load_gather(a_v, [cr, av])
                return acc + qwcol * acol

            exs.append(jnp.exp(alpha * _INV_SQRT_C))
        return exs

    def _msg_all(b, exs):
        for grp in range(_CHUNK // 16):
            rows = rows0 + (grp * 16)
            aoff = b * _CHUNK + grp * 16
            ex = exs[grp]

            av = jnp.full((16,), aoff, jnp.int32) + rows0

            @plsc.parallel_loop(0, _HC, unroll=8)
            def _msg_v_cols(c):
                cr = (jnp.full((16,), c, jnp.int32) + rows0) & (_HC - 1)
                vcol = plsc.load_gather(kv_v[b], [rows, cr + _C])
                plsc.store_scatter(msg_v[b], [rows, cr], ex * vcol)

            @plsc.parallel_loop(0, _AH, unroll=8)
            def _msg_a_cols(c):
                cr = (jnp.full((16,), c, jnp.int32) + rows0) & (_AH - 1)
                acol = plsc.load_gather(a_v, [cr + cid * _AH, av])
                plsc.store_scatter(msg_v[b], [rows, cr + _HC], ex * acol)

            plsc.store_scatter(msg_v[b], [rows, den_col], ex)

    def _fire_qq(b):
        return pltpu.async_copy(
            qq_hbm.at[ei_v.at[1, pl.ds(b * _CHUNK, _CHUNK)]], qq_v[b], sem_q)

    def _fire_kv(b):
        return pltpu.async_copy(
            kvh_hbm.at[cid].at[ei_v.at[0, pl.ds(b * _CHUNK, _CHUNK)]],
            kv_v[b], gsem[b])

    def _fetch_idx(g):
        c1 = pltpu.async_copy(
            ei_hbm.at[:, pl.ds((crow0 + g) * _CHUNK, 2 * _CHUNK)], ei_v,
            sem_i)
        c2 = pltpu.async_copy(
            ea_hbm.at[:, pl.ds((crow0 + g) * _CHUNK, 2 * _CHUNK)], a_v,
            sem_i)
        return c1, c2

    def _stash_and_scatter(b, exs):
        @pl.loop(0, _CHUNK // 16)
        def _cpd(j):
            da_v[b][0, pl.ds(j * 16, 16)] = ei_v[1, pl.ds(b * _CHUNK +
                                                          j * 16, 16)]
        _msg_all(b, exs)
        pltpu.async_copy(msg_v[b], acc_sh.at[da_v[b].at[0]], ssem[b],
                         add=True)

    _fetch_idx(0)

    @pl.loop(0, _NCHUNK, step=2)
    def _iter(g):
        ebase = (crow0 + g) * _CHUNK
        pltpu.make_async_copy(ei_hbm.at[:, pl.ds(ebase, 2 * _CHUNK)], ei_v,
                              sem_i).wait()
        pltpu.make_async_copy(ea_hbm.at[:, pl.ds(ebase, 2 * _CHUNK)], a_v,
                              sem_i).wait()
        kva = _fire_kv(0)
        kvb = _fire_kv(1)
        qqa = _fire_qq(0)

        qqb = _fire_qq(1)
        qqa.wait()
        kva.wait()
        exs = _alpha_all(0)

        @pl.when(g > 0)
        def _():
            pltpu.make_async_copy(msg_v[0], acc_sh.at[da_v[0].at[0]],
                                  ssem[0]).wait()
        _stash_and_scatter(0, exs)

        qqb.wait()
        kvb.wait()
        exs = _alpha_all(1)

        @pl.when(g > 0)
        def _():
            pltpu.make_async_copy(msg_v[1], acc_sh.at[da_v[1].at[0]],
                                  ssem[1]).wait()
        _stash_and_scatter(1, exs)

        @pl.when(g + 2 < _NCHUNK)
        def _():
            _fetch_idx(g + 2)

    for b in range(2):
        pltpu.make_async_copy(msg_v[b], acc_sh.at[da_v[b].at[0]],
                              ssem[b]).wait()

    plsc.subcore_barrier()

    @pl.loop(0, _RPT // _RCHUNK)
    def _wout(t):
        r0 = sid * _RPT + t * _RCHUNK
        pltpu.sync_copy(acc_sh.at[pl.ds(r0, _RCHUNK), :], zbuf)
        pltpu.sync_copy(zbuf, out_hbm.at[cid, pl.ds(r0, _RCHUNK), :])


_sc_mesh = plsc.VectorSubcoreMesh(core_axis_name="c", subcore_axis_name="s",
                                  num_cores=_NC, num_subcores=_NS)

_sc_edge = pl.kernel(
    _sc_edge_body,
    out_type=jax.ShapeDtypeStruct((_NC, _NP, _ACCW), jnp.float32),
    mesh=_sc_mesh,
    compiler_params=pltpu.CompilerParams(use_tc_tiling_on_sc=False,
                                         needs_layout_passes=False),
    scratch_types=[
        pltpu.VMEM((2, 2 * _CHUNK), jnp.int32),
        [pltpu.VMEM((1, _CHUNK), jnp.int32) for _ in range(2)],
        [pltpu.VMEM((_CHUNK, _QQW), jnp.float32) for _ in range(2)],
        [pltpu.VMEM((_CHUNK, _KVW), jnp.float32) for _ in range(2)],
        pltpu.VMEM((_ED, 2 * _CHUNK), jnp.float32),
        [pltpu.VMEM((_CHUNK, _ACCW), jnp.float32) for _ in range(2)],
        pltpu.VMEM((_RCHUNK, _ACCW), jnp.float32),
        pltpu.VMEM_SHARED((_NP, _ACCW), jnp.float32),
        pltpu.SemaphoreType.DMA,
        pltpu.SemaphoreType.DMA,
        pltpu.SemaphoreType.DMA,
        pltpu.SemaphoreType.DMA,
        pltpu.SemaphoreType.DMA,
        pltpu.SemaphoreType.DMA,
    ],
)


@jax.jit
def kernel(x, edge_index, edge_attr,
           Wq0, bq0, Wk0, bk0, Wv0, bv0, We0, Ws0, bs0,
           Wq1, bq1, Wk1, bk1, Wv1, bv1, We1, Ws1, bs1,
           Wlin, blin):
    w0 = jnp.concatenate([Wq0, Wk0, Wv0, Ws0], axis=0)
    b0 = jnp.concatenate([bq0, bk0, bv0, bs0]).reshape(1, -1)
    qq0, kvh0, s0 = _proj(x, w0, b0, We0)
    eat = edge_attr.T
    parts0 = _sc_edge(qq0, kvh0, edge_index, eat)
    h1 = _post1(parts0[0], parts0[1], s0, We0)

    x2 = jnp.concatenate([h1, x], axis=1)
    w1 = jnp.concatenate([Wq1, Wk1, Wv1, Ws1], axis=0)
    b1 = jnp.concatenate([bq1, bk1, bv1, bs1]).reshape(1, -1)
    qq1, kvh1, s1 = _proj(x2, w1, b1, We1)
    parts1 = _sc_edge(qq1, kvh1, edge_index, eat)
    return _post2(parts1[0], parts1[1], s1, We1, Wlin,
                  blin.reshape(1, -1))


# speedup vs baseline: 7.8903x; 1.0753x over previous
import jax
import jax.numpy as jnp
from jax import lax
from jax.experimental import pallas as pl
from jax.experimental.pallas import tpu as pltpu
from jax.experimental.pallas import tpu_sc as plsc

_N = 10000
_E = 320000
_C = 128
_ED = 16
_HC = 64
_NC = 2
_NS = 16
_QQW = _C + _ED
_KVW = _C + _HC
_AH = _ED // 2
_ACCW = 80
_CHUNK = 80
_EPT = _E // _NS
_NCHUNK = _EPT // _CHUNK
_NP = 10240
_RPT = _NP // _NS
_RCHUNK = 32
_INV_SQRT_C = 1.0 / float(_C) ** 0.5


def _proj_body(x_ref, w_ref, b_ref, we_ref, qq_ref, kvh_ref, s_ref):
    z = lax.dot_general(x_ref[...], w_ref[...], (((1,), (1,)), ((), ())),
                        preferred_element_type=jnp.float32)
    z = z + b_ref[...]
    q = z[:, :_C]
    qw = jnp.dot(q, we_ref[...], preferred_element_type=jnp.float32)
    qq_ref[...] = jnp.concatenate([q, qw], axis=1)
    k = z[:, _C:2 * _C]
    v = z[:, 2 * _C:3 * _C]
    kvh_ref[0] = jnp.concatenate([k, v[:, :_HC]], axis=1)
    kvh_ref[1] = jnp.concatenate([k, v[:, _HC:]], axis=1)
    s_ref[...] = z[:, 3 * _C:]


def _proj(x2, wcat_t, bcat, we):
    ind = x2.shape[1]
    blk = 2000
    return pl.pallas_call(
        _proj_body,
        grid=(_N // blk,),
        in_specs=[
            pl.BlockSpec((blk, ind), lambda i: (i, 0)),
            pl.BlockSpec((4 * _C, ind), lambda i: (0, 0)),
            pl.BlockSpec((1, 4 * _C), lambda i: (0, 0)),
            pl.BlockSpec((_C, _ED), lambda i: (0, 0)),
        ],
        out_specs=[
            pl.BlockSpec((blk, _QQW), lambda i: (i, 0)),
            pl.BlockSpec((_NC, blk, _KVW), lambda i: (0, i, 0)),
            pl.BlockSpec((blk, _C), lambda i: (i, 0)),
        ],
        out_shape=[
            jax.ShapeDtypeStruct((_N, _QQW), jnp.float32),
            jax.ShapeDtypeStruct((_NC, _N, _KVW), jnp.float32),
            jax.ShapeDtypeStruct((_N, _C), jnp.float32),
        ],
    )(x2, wcat_t, bcat, we)


def _post_body(a0_ref, a1_ref, s_ref, wet_ref, h_ref):
    num = jnp.concatenate([a0_ref[:, :_HC], a1_ref[:, :_HC]], axis=1)
    t = jnp.concatenate([a0_ref[:, _HC:_HC + _AH],
                         a1_ref[:, _HC:_HC + _AH]], axis=1)
    num = num + lax.dot_general(t, wet_ref[...], (((1,), (1,)), ((), ())),
                                preferred_element_type=jnp.float32)
    den = a0_ref[:, _HC + _AH:_HC + _AH + 1]
    h_ref[...] = num / (den + 1e-16) + s_ref[...]


def _post1(a0, a1, s, wet):
    blk = 2000
    return pl.pallas_call(
        _post_body,
        grid=(_N // blk,),
        in_specs=[
            pl.BlockSpec((blk, _ACCW), lambda i: (i, 0)),
            pl.BlockSpec((blk, _ACCW), lambda i: (i, 0)),
            pl.BlockSpec((blk, _C), lambda i: (i, 0)),
            pl.BlockSpec((_C, _ED), lambda i: (0, 0)),
        ],
        out_specs=pl.BlockSpec((blk, _C), lambda i: (i, 0)),
        out_shape=jax.ShapeDtypeStruct((_N, _C), jnp.float32),
    )(a0, a1, s, wet)


def _post2_body(a0_ref, a1_ref, s_ref, wet_ref, wlt_ref, bl_ref, o_ref):
    num = jnp.concatenate([a0_ref[:, :_HC], a1_ref[:, :_HC]], axis=1)
    t = jnp.concatenate([a0_ref[:, _HC:_HC + _AH],
                         a1_ref[:, _HC:_HC + _AH]], axis=1)
    num = num + lax.dot_general(t, wet_ref[...], (((1,), (1,)), ((), ())),
                                preferred_element_type=jnp.float32)
    den = a0_ref[:, _HC + _AH:_HC + _AH + 1]
    h = num / (den + 1e-16) + s_ref[...]
    o_ref[...] = lax.dot_general(h, wlt_ref[...], (((1,), (1,)), ((), ())),
                                 preferred_element_type=jnp.float32)
    o_ref[...] += bl_ref[...]


def _post2(a0, a1, s, wet, wlt, bl):
    blk = 2000
    return pl.pallas_call(
        _post2_body,
        grid=(_N // blk,),
        in_specs=[
            pl.BlockSpec((blk, _ACCW), lambda i: (i, 0)),
            pl.BlockSpec((blk, _ACCW), lambda i: (i, 0)),
            pl.BlockSpec((blk, _C), lambda i: (i, 0)),
            pl.BlockSpec((_C, _ED), lambda i: (0, 0)),
            pl.BlockSpec((_C, _C), lambda i: (0, 0)),
            pl.BlockSpec((1, _C), lambda i: (0, 0)),
        ],
        out_specs=pl.BlockSpec((blk, _C), lambda i: (i, 0)),
        out_shape=jax.ShapeDtypeStruct((_N, _C), jnp.float32),
    )(a0, a1, s, wet, wlt, bl)


def _sc_edge_body(qq_hbm, kvh_hbm, ei_hbm, ea_hbm, out_hbm,
                  ei_v, da_v, qq_v, kv_v, a_v, msg_v, zbuf, acc_sh,
                  sem_q, sem_g0, sem_g1, sem_s0, sem_s1, sem_i, sem_e):
    cid = lax.axis_index("c")
    sid = lax.axis_index("s")
    zero16 = jnp.zeros((16,), jnp.float32)

    @pl.loop(0, _RCHUNK)
    def _zr(r):
        @pl.loop(0, _ACCW // 16)
        def _zc(c):
            zbuf[r, pl.ds(c * 16, 16)] = zero16

    for b in range(2):
        @pl.loop(0, _CHUNK)
        def _zm(r):
            msg_v[b][r, pl.ds(_HC, 16)] = zero16

    @pl.loop(0, _RPT // _RCHUNK)
    def _zi(t):
        pltpu.sync_copy(zbuf, acc_sh.at[pl.ds(sid * _RPT + t * _RCHUNK,
                                              _RCHUNK), :])

    plsc.subcore_barrier()

    crow0 = sid * _NCHUNK
    rows0 = lax.iota(jnp.int32, 16)
    den_col = jnp.full((16,), _HC + _AH, jnp.int32)
    gsem = [sem_g0, sem_g1]
    ssem = [sem_s0, sem_s1]

    def _alpha_all(b):
        exs = []
        for grp in range(_CHUNK // 16):
            rows = rows0 + (grp * 16)
            aoff = b * _CHUNK + grp * 16

            av = jnp.full((16,), aoff, jnp.int32) + rows0

            @plsc.parallel_loop(0, _C, unroll=8, carry=zero16)
            def alpha(c, acc):
                cv = (jnp.full((16,), c, jnp.int32) + rows0) & (_C - 1)
                qcol = plsc.load_gather(qq_v[b], [rows, cv])
                kcol = plsc.load_gather(kv_v[b], [rows, cv])
                return acc + qcol * kcol

            @plsc.parallel_loop(0, _ED, unroll=8, carry=alpha)
            def alpha(c, acc):
                cr = (jnp.full((16,), c, jnp.int32) + rows0) & (_ED - 1)
                qwcol = plsc.load_gather(qq_v[b], [rows, cr + _C])
                acol = plsc.load_gather(a_v, [cr, av])
                return acc + qwcol * acol

            exs.append(jnp.exp(alpha * _INV_SQRT_C))
        return exs

    def _msg_all(b, exs):
        for grp in range(_CHUNK // 16):
            rows = rows0 + (grp * 16)
            aoff = b * _CHUNK + grp * 16
            ex = exs[grp]

            av = jnp.full((16,), aoff, jnp.int32) + rows0

            @plsc.parallel_loop(0, _HC, unroll=8)
            def _msg_v_cols(c):
                cr = (jnp.full((16,), c, jnp.int32) + rows0) & (_HC - 1)
                vcol = plsc.load_gather(kv_v[b], [rows, cr + _C])
                plsc.store_scatter(msg_v[b], [rows, cr], ex * vcol)

            @plsc.parallel_loop(0, _AH, unroll=8)
            def _msg_a_cols(c):
                cr = (jnp.full((16,), c, jnp.int32) + rows0) & (_AH - 1)
                acol = plsc.load_gather(a_v, [cr + cid * _AH, av])
                plsc.store_scatter(msg_v[b], [rows, cr + _HC], ex * acol)

            plsc.store_scatter(msg_v[b], [rows, den_col], ex)

    def _fire_qq(b):
        return pltpu.async_copy(qq_hbm.at[ei_v[b].at[1]], qq_v[b], sem_q)

    def _fire_kv(b):
        return pltpu.async_copy(kvh_hbm.at[cid].at[ei_v[b].at[0]],
                                kv_v[b], gsem[b])

    def _fire_ei(b, g):
        return pltpu.async_copy(
            ei_hbm.at[:, pl.ds((crow0 + g + b) * _CHUNK, _CHUNK)],
            ei_v[b], sem_i)

    def _fire_ea(g):
        return pltpu.async_copy(
            ea_hbm.at[:, pl.ds((crow0 + g) * _CHUNK, 2 * _CHUNK)], a_v,
            sem_e)

    def _stash(b):
        @pl.loop(0, _CHUNK // 16)
        def _cpd(j):
            da_v[b][0, pl.ds(j * 16, 16)] = ei_v[b][1, pl.ds(j * 16, 16)]

    _fire_ei(0, 0).wait()
    _fire_ei(1, 0).wait()
    _fire_ea(0).wait()
    _fire_kv(0)
    _fire_kv(1)
    _fire_qq(0)
    _fire_qq(1)

    @pl.loop(0, _NCHUNK, step=2)
    def _iter(g):
        for b in range(2):
            @pl.when(g > 0)
            def _():
                pltpu.make_async_copy(msg_v[b], acc_sh.at[da_v[b].at[0]],
                                      ssem[b]).wait()
            _stash(b)

        @pl.when(g > 0)
        def _():
            pltpu.make_async_copy(
                ea_hbm.at[:, pl.ds((crow0 + g) * _CHUNK, 2 * _CHUNK)], a_v,
                sem_e).wait()

        for b in range(2):
            pltpu.make_async_copy(qq_hbm.at[ei_v[b].at[1]], qq_v[b],
                                  sem_q).wait()
            pltpu.make_async_copy(kvh_hbm.at[cid].at[ei_v[b].at[0]],
                                  kv_v[b], gsem[b]).wait()

            @pl.when(g + 2 < _NCHUNK)
            def _():
                _fire_ei(b, g + 2)

            exs = _alpha_all(b)
            _msg_all(b, exs)
            pltpu.async_copy(msg_v[b], acc_sh.at[da_v[b].at[0]], ssem[b],
                             add=True)

        @pl.when(g + 2 < _NCHUNK)
        def _():
            _fire_ea(g + 2)
            pltpu.make_async_copy(
                ei_hbm.at[:, pl.ds((crow0 + g + 2) * _CHUNK, _CHUNK)],
                ei_v[0], sem_i).wait()
            pltpu.make_async_copy(
                ei_hbm.at[:, pl.ds((crow0 + g + 3) * _CHUNK, _CHUNK)],
                ei_v[1], sem_i).wait()
            _fire_kv(0)
            _fire_kv(1)
            _fire_qq(0)
            _fire_qq(1)

    for b in range(2):
        pltpu.make_async_copy(msg_v[b], acc_sh.at[da_v[b].at[0]],
                              ssem[b]).wait()

    plsc.subcore_barrier()

    @pl.loop(0, _RPT // _RCHUNK)
    def _wout(t):
        r0 = sid * _RPT + t * _RCHUNK
        pltpu.sync_copy(acc_sh.at[pl.ds(r0, _RCHUNK), :], zbuf)
        pltpu.sync_copy(zbuf, out_hbm.at[cid, pl.ds(r0, _RCHUNK), :])


_sc_mesh = plsc.VectorSubcoreMesh(core_axis_name="c", subcore_axis_name="s",
                                  num_cores=_NC, num_subcores=_NS)

_sc_edge = pl.kernel(
    _sc_edge_body,
    out_type=jax.ShapeDtypeStruct((_NC, _NP, _ACCW), jnp.float32),
    mesh=_sc_mesh,
    compiler_params=pltpu.CompilerParams(use_tc_tiling_on_sc=False,
                                         needs_layout_passes=False),
    scratch_types=[
        [pltpu.VMEM((2, _CHUNK), jnp.int32) for _ in range(2)],
        [pltpu.VMEM((1, _CHUNK), jnp.int32) for _ in range(2)],
        [pltpu.VMEM((_CHUNK, _QQW), jnp.float32) for _ in range(2)],
        [pltpu.VMEM((_CHUNK, _KVW), jnp.float32) for _ in range(2)],
        pltpu.VMEM((_ED, 2 * _CHUNK), jnp.float32),
        [pltpu.VMEM((_CHUNK, _ACCW), jnp.float32) for _ in range(2)],
        pltpu.VMEM((_RCHUNK, _ACCW), jnp.float32),
        pltpu.VMEM_SHARED((_NP, _ACCW), jnp.float32),
        pltpu.SemaphoreType.DMA,
        pltpu.SemaphoreType.DMA,
        pltpu.SemaphoreType.DMA,
        pltpu.SemaphoreType.DMA,
        pltpu.SemaphoreType.DMA,
        pltpu.SemaphoreType.DMA,
        pltpu.SemaphoreType.DMA,
    ],
)


@jax.jit
def kernel(x, edge_index, edge_attr,
           Wq0, bq0, Wk0, bk0, Wv0, bv0, We0, Ws0, bs0,
           Wq1, bq1, Wk1, bk1, Wv1, bv1, We1, Ws1, bs1,
           Wlin, blin):
    w0 = jnp.concatenate([Wq0, Wk0, Wv0, Ws0], axis=0)
    b0 = jnp.concatenate([bq0, bk0, bv0, bs0]).reshape(1, -1)
    qq0, kvh0, s0 = _proj(x, w0, b0, We0)
    eat = edge_attr.T
    parts0 = _sc_edge(qq0, kvh0, edge_index, eat)
    h1 = _post1(parts0[0], parts0[1], s0, We0)

    x2 = jnp.concatenate([h1, x], axis=1)
    w1 = jnp.concatenate([Wq1, Wk1, Wv1, Ws1], axis=0)
    b1 = jnp.concatenate([bq1, bk1, bv1, bs1]).reshape(1, -1)
    qq1, kvh1, s1 = _proj(x2, w1, b1, We1)
    parts1 = _sc_edge(qq1, kvh1, edge_index, eat)
    return _post2(parts1[0], parts1[1], s1, We1, Wlin,
                  blin.reshape(1, -1))


# speedup vs baseline: 7.9297x; 1.0050x over previous
import jax
import jax.numpy as jnp
from jax import lax
from jax.experimental import pallas as pl
from jax.experimental.pallas import tpu as pltpu
from jax.experimental.pallas import tpu_sc as plsc

_N = 10000
_E = 320000
_C = 128
_ED = 16
_HC = 64
_NC = 2
_NS = 16
_QQW = _C + _ED
_KVW = _C + _HC
_AH = _ED // 2
_ACCW = 80
_CHUNK = 80
_EPT = _E // _NS
_NCHUNK = _EPT // _CHUNK
_NP = 10240
_RPT = _NP // _NS
_RCHUNK = 64
_INV_SQRT_C = 1.0 / float(_C) ** 0.5


def _proj_body(x_ref, w_ref, b_ref, we_ref, qq_ref, kvh_ref, s_ref):
    z = lax.dot_general(x_ref[...], w_ref[...], (((1,), (1,)), ((), ())),
                        preferred_element_type=jnp.float32)
    z = z + b_ref[...]
    q = z[:, :_C]
    qw = jnp.dot(q, we_ref[...], preferred_element_type=jnp.float32)
    qq_ref[...] = jnp.concatenate([q, qw], axis=1)
    k = z[:, _C:2 * _C]
    v = z[:, 2 * _C:3 * _C]
    kvh_ref[0] = jnp.concatenate([k, v[:, :_HC]], axis=1)
    kvh_ref[1] = jnp.concatenate([k, v[:, _HC:]], axis=1)
    s_ref[...] = z[:, 3 * _C:]


def _proj(x2, wcat_t, bcat, we):
    ind = x2.shape[1]
    blk = 2000
    return pl.pallas_call(
        _proj_body,
        grid=(_N // blk,),
        in_specs=[
            pl.BlockSpec((blk, ind), lambda i: (i, 0)),
            pl.BlockSpec((4 * _C, ind), lambda i: (0, 0)),
            pl.BlockSpec((1, 4 * _C), lambda i: (0, 0)),
            pl.BlockSpec((_C, _ED), lambda i: (0, 0)),
        ],
        out_specs=[
            pl.BlockSpec((blk, _QQW), lambda i: (i, 0)),
            pl.BlockSpec((_NC, blk, _KVW), lambda i: (0, i, 0)),
            pl.BlockSpec((blk, _C), lambda i: (i, 0)),
        ],
        out_shape=[
            jax.ShapeDtypeStruct((_N, _QQW), jnp.float32),
            jax.ShapeDtypeStruct((_NC, _N, _KVW), jnp.float32),
            jax.ShapeDtypeStruct((_N, _C), jnp.float32),
        ],
    )(x2, wcat_t, bcat, we)


def _post_body(a0_ref, a1_ref, s_ref, wet_ref, h_ref):
    num = jnp.concatenate([a0_ref[:, :_HC], a1_ref[:, :_HC]], axis=1)
    t = jnp.concatenate([a0_ref[:, _HC:_HC + _AH],
                         a1_ref[:, _HC:_HC + _AH]], axis=1)
    num = num + lax.dot_general(t, wet_ref[...], (((1,), (1,)), ((), ())),
                                preferred_element_type=jnp.float32)
    den = a0_ref[:, _HC + _AH:_HC + _AH + 1]
    h_ref[...] = num / (den + 1e-16) + s_ref[...]


def _post1(a0, a1, s, wet):
    blk = 2000
    return pl.pallas_call(
        _post_body,
        grid=(_N // blk,),
        in_specs=[
            pl.BlockSpec((blk, _ACCW), lambda i: (i, 0)),
            pl.BlockSpec((blk, _ACCW), lambda i: (i, 0)),
            pl.BlockSpec((blk, _C), lambda i: (i, 0)),
            pl.BlockSpec((_C, _ED), lambda i: (0, 0)),
        ],
        out_specs=pl.BlockSpec((blk, _C), lambda i: (i, 0)),
        out_shape=jax.ShapeDtypeStruct((_N, _C), jnp.float32),
    )(a0, a1, s, wet)


def _post2_body(a0_ref, a1_ref, s_ref, wet_ref, wlt_ref, bl_ref, o_ref):
    num = jnp.concatenate([a0_ref[:, :_HC], a1_ref[:, :_HC]], axis=1)
    t = jnp.concatenate([a0_ref[:, _HC:_HC + _AH],
                         a1_ref[:, _HC:_HC + _AH]], axis=1)
    num = num + lax.dot_general(t, wet_ref[...], (((1,), (1,)), ((), ())),
                                preferred_element_type=jnp.float32)
    den = a0_ref[:, _HC + _AH:_HC + _AH + 1]
    h = num / (den + 1e-16) + s_ref[...]
    o_ref[...] = lax.dot_general(h, wlt_ref[...], (((1,), (1,)), ((), ())),
                                 preferred_element_type=jnp.float32)
    o_ref[...] += bl_ref[...]


def _post2(a0, a1, s, wet, wlt, bl):
    blk = 2000
    return pl.pallas_call(
        _post2_body,
        grid=(_N // blk,),
        in_specs=[
            pl.BlockSpec((blk, _ACCW), lambda i: (i, 0)),
            pl.BlockSpec((blk, _ACCW), lambda i: (i, 0)),
            pl.BlockSpec((blk, _C), lambda i: (i, 0)),
            pl.BlockSpec((_C, _ED), lambda i: (0, 0)),
            pl.BlockSpec((_C, _C), lambda i: (0, 0)),
            pl.BlockSpec((1, _C), lambda i: (0, 0)),
        ],
        out_specs=pl.BlockSpec((blk, _C), lambda i: (i, 0)),
        out_shape=jax.ShapeDtypeStruct((_N, _C), jnp.float32),
    )(a0, a1, s, wet, wlt, bl)


def _sc_edge_body(qq_hbm, kvh_hbm, ei_hbm, ea_hbm, out_hbm,
                  ei_v, da_v, qq_v, kv_v, a_v, msg_v, zbuf, acc_sh,
                  sem_q, sem_g0, sem_g1, sem_s0, sem_s1, sem_i, sem_e):
    cid = lax.axis_index("c")
    sid = lax.axis_index("s")
    zero16 = jnp.zeros((16,), jnp.float32)

    @pl.loop(0, _RCHUNK)
    def _zr(r):
        @pl.loop(0, _ACCW // 16)
        def _zc(c):
            zbuf[r, pl.ds(c * 16, 16)] = zero16

    for b in range(2):
        @pl.loop(0, _CHUNK)
        def _zm(r):
            msg_v[b][r, pl.ds(_HC, 16)] = zero16

    @pl.loop(0, _RPT // _RCHUNK)
    def _zi(t):
        pltpu.sync_copy(zbuf, acc_sh.at[pl.ds(sid * _RPT + t * _RCHUNK,
                                              _RCHUNK), :])

    plsc.subcore_barrier()

    crow0 = sid * _NCHUNK
    rows0 = lax.iota(jnp.int32, 16)
    den_col = jnp.full((16,), _HC + _AH, jnp.int32)
    gsem = [sem_g0, sem_g1]
    ssem = [sem_s0, sem_s1]

    def _alpha_all(b):
        exs = []
        for grp in range(_CHUNK // 16):
            rows = rows0 + (grp * 16)
            aoff = b * _CHUNK + grp * 16

            av = jnp.full((16,), aoff, jnp.int32) + rows0

            @plsc.parallel_loop(0, _C, unroll=8, carry=zero16)
            def alpha(c, acc):
                cv = (jnp.full((16,), c, jnp.int32) + rows0) & (_C - 1)
                qcol = plsc.load_gather(qq_v[b], [rows, cv])
                kcol = plsc.load_gather(kv_v[b], [rows, cv])
                return acc + qcol * kcol

            @plsc.parallel_loop(0, _ED, unroll=8, carry=alpha)
            def alpha(c, acc):
                cr = (jnp.full((16,), c, jnp.int32) + rows0) & (_ED - 1)
                qwcol = plsc.load_gather(qq_v[b], [rows, cr + _C])
                acol = plsc.load_gather(a_v, [cr, av])
                return acc + qwcol * acol

            exs.append(jnp.exp(alpha * _INV_SQRT_C))
        return exs

    def _msg_all(b, exs):
        for grp in range(_CHUNK // 16):
            rows = rows0 + (grp * 16)
            aoff = b * _CHUNK + grp * 16
            ex = exs[grp]

            av = jnp.full((16,), aoff, jnp.int32) + rows0

            @plsc.parallel_loop(0, _HC, unroll=8)
            def _msg_v_cols(c):
                cr = (jnp.full((16,), c, jnp.int32) + rows0) & (_HC - 1)
                vcol = plsc.load_gather(kv_v[b], [rows, cr + _C])
                plsc.store_scatter(msg_v[b], [rows, cr], ex * vcol)

            @plsc.parallel_loop(0, _AH, unroll=8)
            def _msg_a_cols(c):
                cr = (jnp.full((16,), c, jnp.int32) + rows0) & (_AH - 1)
                acol = plsc.load_gather(a_v, [cr + cid * _AH, av])
                plsc.store_scatter(msg_v[b], [rows, cr + _HC], ex * acol)

            plsc.store_scatter(msg_v[b], [rows, den_col], ex)

    def _fire_qq(b):
        return pltpu.async_copy(qq_hbm.at[ei_v[b].at[1]], qq_v[b], sem_q)

    def _fire_kv(b):
        return pltpu.async_copy(kvh_hbm.at[cid].at[ei_v[b].at[0]],
                                kv_v[b], gsem[b])

    def _fire_ei(b, g):
        return pltpu.async_copy(
            ei_hbm.at[:, pl.ds((crow0 + g + b) * _CHUNK, _CHUNK)],
            ei_v[b], sem_i)

    def _fire_ea(g):
        return pltpu.async_copy(
            ea_hbm.at[:, pl.ds((crow0 + g) * _CHUNK, 2 * _CHUNK)], a_v,
            sem_e)

    def _stash(b):
        @pl.loop(0, _CHUNK // 16)
        def _cpd(j):
            da_v[b][0, pl.ds(j * 16, 16)] = ei_v[b][1, pl.ds(j * 16, 16)]

    _fire_ei(0, 0).wait()
    _fire_ei(1, 0).wait()
    _fire_ea(0).wait()
    _fire_kv(0)
    _fire_kv(1)
    _fire_qq(0)
    _fire_qq(1)

    @pl.loop(0, _NCHUNK, step=2)
    def _iter(g):
        for b in range(2):
            @pl.when(g > 0)
            def _():
                pltpu.make_async_copy(msg_v[b], acc_sh.at[da_v[b].at[0]],
                                      ssem[b]).wait()
            _stash(b)

        @pl.when(g > 0)
        def _():
            pltpu.make_async_copy(
                ea_hbm.at[:, pl.ds((crow0 + g) * _CHUNK, 2 * _CHUNK)], a_v,
                sem_e).wait()

        for b in range(2):
            pltpu.make_async_copy(qq_hbm.at[ei_v[b].at[1]], qq_v[b],
                                  sem_q).wait()
            pltpu.make_async_copy(kvh_hbm.at[cid].at[ei_v[b].at[0]],
                                  kv_v[b], gsem[b]).wait()

            @pl.when(g + 2 < _NCHUNK)
            def _():
                _fire_ei(b, g + 2)

            exs = _alpha_all(b)
            _msg_all(b, exs)
            pltpu.async_copy(msg_v[b], acc_sh.at[da_v[b].at[0]], ssem[b],
                             add=True)

        @pl.when(g + 2 < _NCHUNK)
        def _():
            _fire_ea(g + 2)
            pltpu.make_async_copy(
                ei_hbm.at[:, pl.ds((crow0 + g + 2) * _CHUNK, _CHUNK)],
                ei_v[0], sem_i).wait()
            pltpu.make_async_copy(
                ei_hbm.at[:, pl.ds((crow0 + g + 3) * _CHUNK, _CHUNK)],
                ei_v[1], sem_i).wait()
            _fire_kv(0)
            _fire_kv(1)
            _fire_qq(0)
            _fire_qq(1)

    for b in range(2):
        pltpu.make_async_copy(msg_v[b], acc_sh.at[da_v[b].at[0]],
                              ssem[b]).wait()

    plsc.subcore_barrier()

    @pl.loop(0, _RPT // _RCHUNK)
    def _wout(t):
        r0 = sid * _RPT + t * _RCHUNK
        pltpu.sync_copy(acc_sh.at[pl.ds(r0, _RCHUNK), :],
                        out_hbm.at[cid, pl.ds(r0, _RCHUNK), :])


_sc_mesh = plsc.VectorSubcoreMesh(core_axis_name="c", subcore_axis_name="s",
                                  num_cores=_NC, num_subcores=_NS)

_sc_edge = pl.kernel(
    _sc_edge_body,
    out_type=jax.ShapeDtypeStruct((_NC, _NP, _ACCW), jnp.float32),
    mesh=_sc_mesh,
    compiler_params=pltpu.CompilerParams(use_tc_tiling_on_sc=False,
                                         needs_layout_passes=False),
    scratch_types=[
        [pltpu.VMEM((2, _CHUNK), jnp.int32) for _ in range(2)],
        [pltpu.VMEM((1, _CHUNK), jnp.int32) for _ in range(2)],
        [pltpu.VMEM((_CHUNK, _QQW), jnp.float32) for _ in range(2)],
        [pltpu.VMEM((_CHUNK, _KVW), jnp.float32) for _ in range(2)],
        pltpu.VMEM((_ED, 2 * _CHUNK), jnp.float32),
        [pltpu.VMEM((_CHUNK, _ACCW), jnp.float32) for _ in range(2)],
        pltpu.VMEM((_RCHUNK, _ACCW), jnp.float32),
        pltpu.VMEM_SHARED((_NP, _ACCW), jnp.float32),
        pltpu.SemaphoreType.DMA,
        pltpu.SemaphoreType.DMA,
        pltpu.SemaphoreType.DMA,
        pltpu.SemaphoreType.DMA,
        pltpu.SemaphoreType.DMA,
        pltpu.SemaphoreType.DMA,
        pltpu.SemaphoreType.DMA,
    ],
)


@jax.jit
def kernel(x, edge_index, edge_attr,
           Wq0, bq0, Wk0, bk0, Wv0, bv0, We0, Ws0, bs0,
           Wq1, bq1, Wk1, bk1, Wv1, bv1, We1, Ws1, bs1,
           Wlin, blin):
    w0 = jnp.concatenate([Wq0, Wk0, Wv0, Ws0], axis=0)
    b0 = jnp.concatenate([bq0, bk0, bv0, bs0]).reshape(1, -1)
    qq0, kvh0, s0 = _proj(x, w0, b0, We0)
    eat = edge_attr.T
    parts0 = _sc_edge(qq0, kvh0, edge_index, eat)
    h1 = _post1(parts0[0], parts0[1], s0, We0)

    x2 = jnp.concatenate([h1, x], axis=1)
    w1 = jnp.concatenate([Wq1, Wk1, Wv1, Ws1], axis=0)
    b1 = jnp.concatenate([bq1, bk1, bv1, bs1]).reshape(1, -1)
    qq1, kvh1, s1 = _proj(x2, w1, b1, We1)
    parts1 = _sc_edge(qq1, kvh1, edge_index, eat)
    return _post2(parts1[0], parts1[1], s1, We1, Wlin,
                  blin.reshape(1, -1))
